# Initial kernel scaffold; baseline (speedup 1.0000x reference)
import jax
import jax.numpy as jnp
from jax.experimental import pallas as pl


def kernel(x, edge_index, p0, p1, p2, Wd0, Wd1, Wd2, bd0, bd1, bd2, Wu0, Wu1, bu0, bu1):
    raise NotImplementedError("write your pallas kernel here")


# speedup vs baseline: 1.0124x; 1.0124x over previous
import functools
import math

import jax
import jax.numpy as jnp
from jax.experimental import pallas as pl
from jax.experimental.pallas import tpu as pltpu


def _rup(n, m):
    return ((n + m - 1) // m) * m


def _blk(p):
    return 512 if p % 512 == 0 else 256


def _mm_nt_body(u_ref, v_ref, o_ref, acc_ref, *, ksteps, n_real, bm, bn, zero_diag):
    @pl.when(pl.program_id(2) == 0)
    def _():
        acc_ref[...] = jnp.zeros_like(acc_ref)

    acc_ref[...] += jax.lax.dot_general(
        u_ref[...], v_ref[...], (((1,), (1,)), ((), ())),
        preferred_element_type=jnp.float32)

    @pl.when(pl.program_id(2) == ksteps - 1)
    def _():
        mi = pl.program_id(0)
        nj = pl.program_id(1)
        rows = mi * bm + jax.lax.broadcasted_iota(jnp.int32, (bm, bn), 0)
        cols = nj * bn + jax.lax.broadcasted_iota(jnp.int32, (bm, bn), 1)
        mask = (rows < n_real) & (cols < n_real)
        if zero_diag:
            mask &= rows != cols
        o_ref[...] = jnp.where(mask, acc_ref[...], 0.0)


def _mm_nt(u, vt, n_real, zero_diag=True):
    m, k = u.shape
    n = vt.shape[0]
    bm, bn, bk = _blk(m), _blk(n), _blk(k)
    grid = (m // bm, n // bn, k // bk)
    return pl.pallas_call(
        functools.partial(_mm_nt_body, ksteps=grid[2], n_real=n_real,
                          bm=bm, bn=bn, zero_diag=zero_diag),
        grid=grid,
        in_specs=[
            pl.BlockSpec((bm, bk), lambda i, j, kk: (i, kk)),
            pl.BlockSpec((bn, bk), lambda i, j, kk: (j, kk)),
        ],
        out_specs=pl.BlockSpec((bm, bn), lambda i, j, kk: (i, j)),
        out_shape=jax.ShapeDtypeStruct((m, n), jnp.float32),
        scratch_shapes=[pltpu.VMEM((bm, bn), jnp.float32)],
        compiler_params=pltpu.CompilerParams(
            dimension_semantics=("parallel", "parallel", "arbitrary")),
    )(u, vt)


def _rowsum_body(a_ref, o_ref, *, nsteps, c):
    @pl.when(pl.program_id(1) == 0)
    def _():
        o_ref[...] = jnp.zeros_like(o_ref)

    s = jnp.sum(a_ref[...], axis=1, keepdims=True)
    o_ref[...] += jnp.broadcast_to(s, o_ref.shape)

    @pl.when(pl.program_id(1) == nsteps - 1)
    def _():
        o_ref[...] += 2.0


def _rowsum(a, c):
    m, n = a.shape
    bm, bn = _blk(m), _blk(n)
    grid = (m // bm, n // bn)
    return pl.pallas_call(
        functools.partial(_rowsum_body, nsteps=grid[1], c=c),
        grid=grid,
        in_specs=[pl.BlockSpec((bm, bn), lambda i, j: (i, j))],
        out_specs=pl.BlockSpec((bm, c), lambda i, j: (i, 0)),
        out_shape=jax.ShapeDtypeStruct((m, c), jnp.float32),
        compiler_params=pltpu.CompilerParams(
            dimension_semantics=("parallel", "arbitrary")),
    )(a)


def _zscale_body(inp_ref, w_ref, deg_ref, o_ref):
    xw = jnp.dot(inp_ref[...], w_ref[...], preferred_element_type=jnp.float32)
    o_ref[...] = xw * jax.lax.rsqrt(deg_ref[...])


def _zscale(inp, w, deg):
    m, c = inp.shape
    bm = _blk(m)
    grid = (m // bm,)
    return pl.pallas_call(
        _zscale_body,
        grid=grid,
        in_specs=[
            pl.BlockSpec((bm, c), lambda i: (i, 0)),
            pl.BlockSpec((c, c), lambda i: (0, 0)),
            pl.BlockSpec((bm, c), lambda i: (i, 0)),
        ],
        out_specs=pl.BlockSpec((bm, c), lambda i: (i, 0)),
        out_shape=jax.ShapeDtypeStruct((m, c), jnp.float32),
    )(inp, w, deg)


def _agg_body(a_ref, z_ref, zr_ref, deg_ref, b_ref, o_ref, acc_ref, *,
              ksteps, n_real, bm):
    @pl.when(pl.program_id(1) == 0)
    def _():
        acc_ref[...] = jnp.zeros_like(acc_ref)

    acc_ref[...] += jnp.dot(a_ref[...], z_ref[...],
                            preferred_element_type=jnp.float32)

    @pl.when(pl.program_id(1) == ksteps - 1)
    def _():
        dinv = jax.lax.rsqrt(deg_ref[...])
        h = dinv * (acc_ref[...] + 2.0 * zr_ref[...]) + b_ref[...]
        rows = pl.program_id(0) * bm + jax.lax.broadcasted_iota(
            jnp.int32, h.shape, 0)
        o_ref[...] = jnp.where(rows < n_real, jnp.maximum(h, 0.0), 0.0)


def _agg(a, z, deg, b, n_real):
    m, k = a.shape
    c = z.shape[1]
    bm, bk = _blk(m), _blk(k)
    grid = (m // bm, k // bk)
    return pl.pallas_call(
        functools.partial(_agg_body, ksteps=grid[1], n_real=n_real, bm=bm),
        grid=grid,
        in_specs=[
            pl.BlockSpec((bm, bk), lambda i, kk: (i, kk)),
            pl.BlockSpec((bk, c), lambda i, kk: (kk, 0)),
            pl.BlockSpec((bm, c), lambda i, kk: (i, 0)),
            pl.BlockSpec((bm, c), lambda i, kk: (i, 0)),
            pl.BlockSpec((1, c), lambda i, kk: (0, 0)),
        ],
        out_specs=pl.BlockSpec((bm, c), lambda i, kk: (i, 0)),
        out_shape=jax.ShapeDtypeStruct((m, c), jnp.float32),
        scratch_shapes=[pltpu.VMEM((bm, c), jnp.float32)],
        compiler_params=pltpu.CompilerParams(
            dimension_semantics=("parallel", "arbitrary")),
    )(a, z, z, deg, b.reshape(1, c))


def _select(score, k):
    vals = jax.lax.top_k(score, k)[0]
    t = vals[k - 1]
    gt = score > t
    ngt = jnp.sum(gt.astype(jnp.int32))
    eq = score == t
    tie = eq & (jnp.cumsum(eq.astype(jnp.int32)) <= (k - ngt))
    sel = gt | tie
    rank = (jnp.cumsum(sel.astype(jnp.int32)) - 1).astype(jnp.int32)
    perm = jnp.nonzero(sel, size=k)[0].astype(jnp.int32)
    return sel, rank, perm, score[perm]


def _pad_rows(a, p):
    return jnp.pad(a, ((0, p - a.shape[0]), (0, 0)))


def _gather_factors(am, perm, kp, n_real_src):
    k = perm.shape[0]
    u = _pad_rows(am[perm, :], kp)
    u = u.at[jnp.arange(k), perm].add(1.0)
    vt = _pad_rows(am[:, perm].T, kp)
    vt = vt.at[jnp.arange(k), perm].add(1.0)
    return u, vt


def kernel(x, edge_index, p0, p1, p2, Wd0, Wd1, Wd2, bd0, bd1, bd2,
           Wu0, Wu1, bu0, bu1):
    n0, c = x.shape
    k1 = -(-n0 // 2)
    k2 = -(-k1 // 2)
    k3 = -(-k2 // 2)
    P0, P1, P2, P3 = (_rup(v, 256) for v in (n0, k1, k2, k3))

    src, dst = edge_index[0], edge_index[1]

    score0 = (x @ p0) / jnp.linalg.norm(p0)
    sel0, rank0, perm0, vals0 = _select(score0, k1)

    r_u = jnp.where(sel0[src], rank0[src], P1 - 1)
    c_u = jnp.where(sel0[src], dst, P0 - 1)
    u0 = jnp.zeros((P1, P0), jnp.float32).at[r_u, c_u].add(1.0)
    u0 = u0.at[jnp.arange(k1), perm0].add(1.0)
    r_v = jnp.where(sel0[dst], rank0[dst], P1 - 1)
    c_v = jnp.where(sel0[dst], src, P0 - 1)
    vt0 = jnp.zeros((P1, P0), jnp.float32).at[r_v, c_v].add(1.0)
    vt0 = vt0.at[jnp.arange(k1), perm0].add(1.0)

    a1 = _mm_nt(u0, vt0, k1)
    deg1 = _rowsum(a1, c)
    xg1 = _pad_rows(x[perm0] * jnp.tanh(vals0)[:, None], P1)
    h1 = _agg(a1, _zscale(xg1, Wd0, deg1), deg1, bd0, k1)

    score1 = (h1[:k1] @ p1) / jnp.linalg.norm(p1)
    sel1, rank1, perm1, vals1 = _select(score1, k2)
    u1, vt1 = _gather_factors(a1, perm1, P2, k1)
    a2 = _mm_nt(u1, vt1, k2)
    deg2 = _rowsum(a2, c)
    xg2 = _pad_rows(h1[perm1] * jnp.tanh(vals1)[:, None], P2)
    h2 = _agg(a2, _zscale(xg2, Wd1, deg2), deg2, bd1, k2)

    score2 = (h2[:k2] @ p2) / jnp.linalg.norm(p2)
    sel2, rank2, perm2, vals2 = _select(score2, k3)
    u2, vt2 = _gather_factors(a2, perm2, P3, k2)
    a3 = _mm_nt(u2, vt2, k3)
    deg3 = _rowsum(a3, c)
    xg3 = _pad_rows(h2[perm2] * jnp.tanh(vals2)[:, None], P3)
    h3 = _agg(a3, _zscale(xg3, Wd2, deg3), deg3, bd2, k3)

    up2 = jnp.zeros((k2, c), jnp.float32).at[perm2].set(h3[:k3])
    r2 = _pad_rows(h2[:k2] + up2, P2)
    g2 = _agg(a2, _zscale(r2, Wu0, deg2), deg2, bu0, k2)

    up1 = jnp.zeros((k1, c), jnp.float32).at[perm1].set(g2[:k2])
    r1 = _pad_rows(h1[:k1] + up1, P1)
    g1 = _agg(a1, _zscale(r1, Wu1, deg1), deg1, bu1, k1)

    out = x + jnp.zeros((n0, c), jnp.float32).at[perm0].set(g1[:k1])
    return out


# speedup vs baseline: 1.0681x; 1.0550x over previous
import functools
import math

import jax
import jax.numpy as jnp
from jax.experimental import pallas as pl
from jax.experimental.pallas import tpu as pltpu


def _rup(n, m):
    return ((n + m - 1) // m) * m


def _blk(p):
    return 512 if p % 512 == 0 else 256


def _mm_nt_body(u_ref, v_ref, o_ref, ot_ref, acc_ref, *, ksteps, n_real, bm,
                bn, zero_diag, want_t):
    @pl.when(pl.program_id(2) == 0)
    def _():
        acc_ref[...] = jnp.zeros_like(acc_ref)

    acc_ref[...] += jax.lax.dot_general(
        u_ref[...], v_ref[...], (((1,), (1,)), ((), ())),
        preferred_element_type=jnp.float32)

    @pl.when(pl.program_id(2) == ksteps - 1)
    def _():
        mi = pl.program_id(0)
        nj = pl.program_id(1)
        rows = mi * bm + jax.lax.broadcasted_iota(jnp.int32, (bm, bn), 0)
        cols = nj * bn + jax.lax.broadcasted_iota(jnp.int32, (bm, bn), 1)
        mask = (rows < n_real) & (cols < n_real)
        if zero_diag:
            mask &= rows != cols
        res = jnp.where(mask, acc_ref[...], 0.0)
        o_ref[...] = res
        ot_ref[...] = res.T


def _mm_nt(u, vt, n_real, zero_diag=True):
    m, k = u.shape
    n = vt.shape[0]
    bm, bn, bk = _blk(m), _blk(n), _blk(k)
    grid = (m // bm, n // bn, k // bk)
    return pl.pallas_call(
        functools.partial(_mm_nt_body, ksteps=grid[2], n_real=n_real,
                          bm=bm, bn=bn, zero_diag=zero_diag, want_t=True),
        grid=grid,
        in_specs=[
            pl.BlockSpec((bm, bk), lambda i, j, kk: (i, kk)),
            pl.BlockSpec((bn, bk), lambda i, j, kk: (j, kk)),
        ],
        out_specs=[pl.BlockSpec((bm, bn), lambda i, j, kk: (i, j)),
                   pl.BlockSpec((bn, bm), lambda i, j, kk: (j, i))],
        out_shape=[jax.ShapeDtypeStruct((m, n), jnp.float32),
                   jax.ShapeDtypeStruct((n, m), jnp.float32)],
        scratch_shapes=[pltpu.VMEM((bm, bn), jnp.float32)],
        compiler_params=pltpu.CompilerParams(
            dimension_semantics=("parallel", "parallel", "arbitrary")),
    )(u, vt)


def _rowsum_body(a_ref, o_ref, *, nsteps, c):
    @pl.when(pl.program_id(1) == 0)
    def _():
        o_ref[...] = jnp.zeros_like(o_ref)

    s = jnp.sum(a_ref[...], axis=1, keepdims=True)
    o_ref[...] += jnp.broadcast_to(s, o_ref.shape)

    @pl.when(pl.program_id(1) == nsteps - 1)
    def _():
        o_ref[...] += 2.0


def _rowsum(a, c):
    m, n = a.shape
    bm, bn = _blk(m), _blk(n)
    grid = (m // bm, n // bn)
    return pl.pallas_call(
        functools.partial(_rowsum_body, nsteps=grid[1], c=c),
        grid=grid,
        in_specs=[pl.BlockSpec((bm, bn), lambda i, j: (i, j))],
        out_specs=pl.BlockSpec((bm, c), lambda i, j: (i, 0)),
        out_shape=jax.ShapeDtypeStruct((m, c), jnp.float32),
        compiler_params=pltpu.CompilerParams(
            dimension_semantics=("parallel", "arbitrary")),
    )(a)


def _zscale_body(inp_ref, w_ref, deg_ref, o_ref):
    xw = jnp.dot(inp_ref[...], w_ref[...], preferred_element_type=jnp.float32)
    o_ref[...] = xw * jax.lax.rsqrt(deg_ref[...])


def _zscale(inp, w, deg):
    m, c = inp.shape
    bm = _blk(m)
    grid = (m // bm,)
    return pl.pallas_call(
        _zscale_body,
        grid=grid,
        in_specs=[
            pl.BlockSpec((bm, c), lambda i: (i, 0)),
            pl.BlockSpec((c, c), lambda i: (0, 0)),
            pl.BlockSpec((bm, c), lambda i: (i, 0)),
        ],
        out_specs=pl.BlockSpec((bm, c), lambda i: (i, 0)),
        out_shape=jax.ShapeDtypeStruct((m, c), jnp.float32),
    )(inp, w, deg)


def _agg_body(a_ref, z_ref, zr_ref, deg_ref, b_ref, o_ref, acc_ref, *,
              ksteps, n_real, bm):
    @pl.when(pl.program_id(1) == 0)
    def _():
        acc_ref[...] = jnp.zeros_like(acc_ref)

    acc_ref[...] += jnp.dot(a_ref[...], z_ref[...],
                            preferred_element_type=jnp.float32)

    @pl.when(pl.program_id(1) == ksteps - 1)
    def _():
        dinv = jax.lax.rsqrt(deg_ref[...])
        h = dinv * (acc_ref[...] + 2.0 * zr_ref[...]) + b_ref[...]
        rows = pl.program_id(0) * bm + jax.lax.broadcasted_iota(
            jnp.int32, h.shape, 0)
        o_ref[...] = jnp.where(rows < n_real, jnp.maximum(h, 0.0), 0.0)


def _agg(a, z, deg, b, n_real):
    m, k = a.shape
    c = z.shape[1]
    bm, bk = _blk(m), _blk(k)
    grid = (m // bm, k // bk)
    return pl.pallas_call(
        functools.partial(_agg_body, ksteps=grid[1], n_real=n_real, bm=bm),
        grid=grid,
        in_specs=[
            pl.BlockSpec((bm, bk), lambda i, kk: (i, kk)),
            pl.BlockSpec((bk, c), lambda i, kk: (kk, 0)),
            pl.BlockSpec((bm, c), lambda i, kk: (i, 0)),
            pl.BlockSpec((bm, c), lambda i, kk: (i, 0)),
            pl.BlockSpec((1, c), lambda i, kk: (0, 0)),
        ],
        out_specs=pl.BlockSpec((bm, c), lambda i, kk: (i, 0)),
        out_shape=jax.ShapeDtypeStruct((m, c), jnp.float32),
        scratch_shapes=[pltpu.VMEM((bm, c), jnp.float32)],
        compiler_params=pltpu.CompilerParams(
            dimension_semantics=("parallel", "arbitrary")),
    )(a, z, z, deg, b.reshape(1, c))


def _select(score, k):
    vals = jax.lax.top_k(score, k)[0]
    t = vals[k - 1]
    gt = score > t
    ngt = jnp.sum(gt.astype(jnp.int32))
    eq = score == t
    tie = eq & (jnp.cumsum(eq.astype(jnp.int32)) <= (k - ngt))
    sel = gt | tie
    rank = (jnp.cumsum(sel.astype(jnp.int32)) - 1).astype(jnp.int32)
    perm = jnp.nonzero(sel, size=k)[0].astype(jnp.int32)
    return sel, rank, perm, score[perm]


def _pad_rows(a, p):
    return jnp.pad(a, ((0, p - a.shape[0]), (0, 0)))


def _gather_factors(am, amt, perm, kp):
    k = perm.shape[0]
    u = _pad_rows(am[perm, :], kp)
    u = u.at[jnp.arange(k), perm].add(1.0)
    vt = _pad_rows(amt[perm, :], kp)
    vt = vt.at[jnp.arange(k), perm].add(1.0)
    return u, vt


def _unpool(res, hnext, sel, rank):
    g = hnext[jnp.where(sel, rank, 0)]
    return res + jnp.where(sel[:, None], g, 0.0)


def kernel(x, edge_index, p0, p1, p2, Wd0, Wd1, Wd2, bd0, bd1, bd2,
           Wu0, Wu1, bu0, bu1):
    n0, c = x.shape
    k1 = -(-n0 // 2)
    k2 = -(-k1 // 2)
    k3 = -(-k2 // 2)
    P0, P1, P2, P3 = (_rup(v, 256) for v in (n0, k1, k2, k3))

    src, dst = edge_index[0], edge_index[1]

    score0 = (x @ p0) / jnp.linalg.norm(p0)
    sel0, rank0, perm0, vals0 = _select(score0, k1)

    r_u = jnp.where(sel0[src], rank0[src], P1 - 1)
    c_u = jnp.where(sel0[src], dst, P0 - 1)
    u0 = jnp.zeros((P1, P0), jnp.float32).at[r_u, c_u].add(1.0)
    u0 = u0.at[jnp.arange(k1), perm0].add(1.0)
    r_v = jnp.where(sel0[dst], rank0[dst], P1 - 1)
    c_v = jnp.where(sel0[dst], src, P0 - 1)
    vt0 = jnp.zeros((P1, P0), jnp.float32).at[r_v, c_v].add(1.0)
    vt0 = vt0.at[jnp.arange(k1), perm0].add(1.0)

    a1, a1t = _mm_nt(u0, vt0, k1)
    deg1 = _rowsum(a1, c)
    xg1 = _pad_rows(x[perm0] * jnp.tanh(vals0)[:, None], P1)
    h1 = _agg(a1, _zscale(xg1, Wd0, deg1), deg1, bd0, k1)

    score1 = (h1[:k1] @ p1) / jnp.linalg.norm(p1)
    sel1, rank1, perm1, vals1 = _select(score1, k2)
    u1, vt1 = _gather_factors(a1, a1t, perm1, P2)
    a2, a2t = _mm_nt(u1, vt1, k2)
    deg2 = _rowsum(a2, c)
    xg2 = _pad_rows(h1[perm1] * jnp.tanh(vals1)[:, None], P2)
    h2 = _agg(a2, _zscale(xg2, Wd1, deg2), deg2, bd1, k2)

    score2 = (h2[:k2] @ p2) / jnp.linalg.norm(p2)
    sel2, rank2, perm2, vals2 = _select(score2, k3)
    u2, vt2 = _gather_factors(a2, a2t, perm2, P3)
    a3, _ = _mm_nt(u2, vt2, k3)
    deg3 = _rowsum(a3, c)
    xg3 = _pad_rows(h2[perm2] * jnp.tanh(vals2)[:, None], P3)
    h3 = _agg(a3, _zscale(xg3, Wd2, deg3), deg3, bd2, k3)

    r2 = _pad_rows(_unpool(h2[:k2], h3[:k3], sel2, rank2), P2)
    g2 = _agg(a2, _zscale(r2, Wu0, deg2), deg2, bu0, k2)

    r1 = _pad_rows(_unpool(h1[:k1], g2[:k2], sel1, rank1), P1)
    g1 = _agg(a1, _zscale(r1, Wu1, deg1), deg1, bu1, k1)

    return _unpool(x, g1[:k1], sel0, rank0)


# speedup vs baseline: 1.0808x; 1.0119x over previous
import functools
import math

import jax
import jax.numpy as jnp
from jax import lax
from jax.experimental import pallas as pl
from jax.experimental.pallas import tpu as pltpu
from jax.experimental.pallas import tpu_sc as plsc


def _rup(n, m):
    return ((n + m - 1) // m) * m


def _blk(p):
    return 512 if p % 512 == 0 else 256


def _mm_nt_body(u_ref, v_ref, o_ref, ot_ref, acc_ref, *, ksteps, n_real, bm,
                bn, want_t):
    @pl.when(pl.program_id(2) == 0)
    def _():
        acc_ref[...] = jnp.zeros_like(acc_ref)

    acc_ref[...] += jax.lax.dot_general(
        u_ref[...], v_ref[...], (((1,), (1,)), ((), ())),
        preferred_element_type=jnp.float32)

    @pl.when(pl.program_id(2) == ksteps - 1)
    def _():
        mi = pl.program_id(0)
        nj = pl.program_id(1)
        rows = mi * bm + jax.lax.broadcasted_iota(jnp.int32, (bm, bn), 0)
        cols = nj * bn + jax.lax.broadcasted_iota(jnp.int32, (bm, bn), 1)
        valid = (rows < n_real) & (cols < n_real)
        res = jnp.where(valid & (rows != cols), acc_ref[...], 0.0)
        res = res + jnp.where(valid & (rows == cols), 1.0, 0.0)
        o_ref[...] = res
        ot_ref[...] = res.T


def _mm_nt(u, vt, n_real):
    m, k = u.shape
    n = vt.shape[0]
    bm, bn, bk = _blk(m), _blk(n), _blk(k)
    grid = (m // bm, n // bn, k // bk)
    return pl.pallas_call(
        functools.partial(_mm_nt_body, ksteps=grid[2], n_real=n_real,
                          bm=bm, bn=bn, want_t=True),
        grid=grid,
        in_specs=[
            pl.BlockSpec((bm, bk), lambda i, j, kk: (i, kk)),
            pl.BlockSpec((bn, bk), lambda i, j, kk: (j, kk)),
        ],
        out_specs=[pl.BlockSpec((bm, bn), lambda i, j, kk: (i, j)),
                   pl.BlockSpec((bn, bm), lambda i, j, kk: (j, i))],
        out_shape=[jax.ShapeDtypeStruct((m, n), jnp.float32),
                   jax.ShapeDtypeStruct((n, m), jnp.float32)],
        scratch_shapes=[pltpu.VMEM((bm, bn), jnp.float32)],
        compiler_params=pltpu.CompilerParams(
            dimension_semantics=("parallel", "parallel", "arbitrary")),
    )(u, vt)


def _rowsum_body(a_ref, o_ref, *, nsteps, c):
    @pl.when(pl.program_id(1) == 0)
    def _():
        o_ref[...] = jnp.zeros_like(o_ref)

    s = jnp.sum(a_ref[...], axis=1, keepdims=True)
    o_ref[...] += jnp.broadcast_to(s, o_ref.shape)

    @pl.when(pl.program_id(1) == nsteps - 1)
    def _():
        o_ref[...] += 1.0


def _rowsum(a, c):
    m, n = a.shape
    bm, bn = _blk(m), _blk(n)
    grid = (m // bm, n // bn)
    return pl.pallas_call(
        functools.partial(_rowsum_body, nsteps=grid[1], c=c),
        grid=grid,
        in_specs=[pl.BlockSpec((bm, bn), lambda i, j: (i, j))],
        out_specs=pl.BlockSpec((bm, c), lambda i, j: (i, 0)),
        out_shape=jax.ShapeDtypeStruct((m, c), jnp.float32),
        compiler_params=pltpu.CompilerParams(
            dimension_semantics=("parallel", "arbitrary")),
    )(a)


def _zscale_body(inp_ref, w_ref, deg_ref, o_ref):
    xw = jnp.dot(inp_ref[...], w_ref[...], preferred_element_type=jnp.float32)
    o_ref[...] = xw * jax.lax.rsqrt(deg_ref[...])


def _zscale(inp, w, deg):
    m, c = inp.shape
    bm = _blk(m)
    grid = (m // bm,)
    return pl.pallas_call(
        _zscale_body,
        grid=grid,
        in_specs=[
            pl.BlockSpec((bm, c), lambda i: (i, 0)),
            pl.BlockSpec((c, c), lambda i: (0, 0)),
            pl.BlockSpec((bm, c), lambda i: (i, 0)),
        ],
        out_specs=pl.BlockSpec((bm, c), lambda i: (i, 0)),
        out_shape=jax.ShapeDtypeStruct((m, c), jnp.float32),
    )(inp, w, deg)


def _agg_body(a_ref, z_ref, zr_ref, deg_ref, b_ref, o_ref, acc_ref, *,
              ksteps, n_real, bm):
    @pl.when(pl.program_id(1) == 0)
    def _():
        acc_ref[...] = jnp.zeros_like(acc_ref)

    acc_ref[...] += jnp.dot(a_ref[...], z_ref[...],
                            preferred_element_type=jnp.float32)

    @pl.when(pl.program_id(1) == ksteps - 1)
    def _():
        dinv = jax.lax.rsqrt(deg_ref[...])
        h = dinv * (acc_ref[...] + zr_ref[...]) + b_ref[...]
        rows = pl.program_id(0) * bm + jax.lax.broadcasted_iota(
            jnp.int32, h.shape, 0)
        o_ref[...] = jnp.where(rows < n_real, jnp.maximum(h, 0.0), 0.0)


def _agg(a, z, deg, b, n_real):
    m, k = a.shape
    c = z.shape[1]
    bm, bk = _blk(m), _blk(k)
    grid = (m // bm, k // bk)
    return pl.pallas_call(
        functools.partial(_agg_body, ksteps=grid[1], n_real=n_real, bm=bm),
        grid=grid,
        in_specs=[
            pl.BlockSpec((bm, bk), lambda i, kk: (i, kk)),
            pl.BlockSpec((bk, c), lambda i, kk: (kk, 0)),
            pl.BlockSpec((bm, c), lambda i, kk: (i, 0)),
            pl.BlockSpec((bm, c), lambda i, kk: (i, 0)),
            pl.BlockSpec((1, c), lambda i, kk: (0, 0)),
        ],
        out_specs=pl.BlockSpec((bm, c), lambda i, kk: (i, 0)),
        out_shape=jax.ShapeDtypeStruct((m, c), jnp.float32),
        scratch_shapes=[pltpu.VMEM((bm, c), jnp.float32)],
        compiler_params=pltpu.CompilerParams(
            dimension_semantics=("parallel", "arbitrary")),
    )(a, z, z, deg, b.reshape(1, c))


def _sc_gather(table, idx, out_rows):
    t_rows, d = table.shape
    nw = 32
    rpw = out_rows // nw
    assert out_rows % nw == 0
    cap = max(8, (384 * 1024) // (d * 4) // 8 * 8)
    c_rows = min(rpw, cap)
    while rpw % c_rows:
        c_rows -= 8
    nchunks = rpw // c_rows
    idx_buf = max(16, c_rows)
    mesh = plsc.VectorSubcoreMesh(core_axis_name="c", subcore_axis_name="s")

    @functools.partial(
        pl.kernel, mesh=mesh,
        out_type=jax.ShapeDtypeStruct((out_rows, d), jnp.float32),
        scratch_types=[
            pltpu.VMEM((idx_buf,), jnp.int32),
            pltpu.VMEM((c_rows, d), jnp.float32),
            pltpu.SemaphoreType.DMA,
        ],
    )
    def k(table_hbm, idx_hbm, out_hbm, idx_v, rows_v, sem):
        wid = lax.axis_index("s") * 2 + lax.axis_index("c")
        for j in range(nchunks):
            base = wid * rpw + j * c_rows
            pltpu.sync_copy(idx_hbm.at[pl.ds(base, c_rows)],
                            idx_v.at[pl.ds(0, c_rows)])
            pltpu.async_copy(table_hbm.at[idx_v.at[pl.ds(0, c_rows)]],
                             rows_v, sem).wait()
            pltpu.sync_copy(rows_v, out_hbm.at[pl.ds(base, c_rows)])

    return k(table, idx)


def _pad_idx(perm, out_rows, zero_row):
    k = perm.shape[0]
    return jnp.concatenate(
        [perm, jnp.full((out_rows - k,), zero_row, jnp.int32)])


def _select(score, k):
    vals = jax.lax.top_k(score, k)[0]
    t = vals[k - 1]
    gt = score > t
    ngt = jnp.sum(gt.astype(jnp.int32))
    eq = score == t
    tie = eq & (jnp.cumsum(eq.astype(jnp.int32)) <= (k - ngt))
    sel = gt | tie
    rank = (jnp.cumsum(sel.astype(jnp.int32)) - 1).astype(jnp.int32)
    perm = jnp.nonzero(sel, size=k)[0].astype(jnp.int32)
    return sel, rank, perm, score[perm]


def _pad_rows(a, p):
    return jnp.pad(a, ((0, p - a.shape[0]), (0, 0)))


def _gather_factors(am, amt, perm, kp):
    k = perm.shape[0]
    psrc = am.shape[0]
    idx = _pad_idx(perm, kp, psrc - 1)
    u = _sc_gather(am, idx, kp)
    vt = _sc_gather(amt, idx, kp)
    return u, vt


def _unpool(res, hnext, sel, rank, out_rows):
    pn = hnext.shape[0]
    idx = jnp.where(sel, rank, pn - 1).astype(jnp.int32)
    idx = _pad_idx(idx, out_rows, pn - 1)
    up = _sc_gather(hnext, idx, out_rows)
    return res + up[: res.shape[0]]


def kernel(x, edge_index, p0, p1, p2, Wd0, Wd1, Wd2, bd0, bd1, bd2,
           Wu0, Wu1, bu0, bu1):
    n0, c = x.shape
    k1 = -(-n0 // 2)
    k2 = -(-k1 // 2)
    k3 = -(-k2 // 2)
    P0, P1, P2, P3 = (_rup(v, 256) for v in (n0, k1, k2, k3))

    src, dst = edge_index[0], edge_index[1]

    score0 = (x @ p0) / jnp.linalg.norm(p0)
    sel0, rank0, perm0, vals0 = _select(score0, k1)

    r_u = jnp.where(sel0[src], rank0[src], P1 - 1)
    c_u = jnp.where(sel0[src], dst, P0 - 1)
    u0 = jnp.zeros((P1, P0), jnp.float32).at[r_u, c_u].add(1.0)
    u0 = u0.at[jnp.arange(k1), perm0].add(1.0)
    r_v = jnp.where(sel0[dst], rank0[dst], P1 - 1)
    c_v = jnp.where(sel0[dst], src, P0 - 1)
    vt0 = jnp.zeros((P1, P0), jnp.float32).at[r_v, c_v].add(1.0)
    vt0 = vt0.at[jnp.arange(k1), perm0].add(1.0)

    a1, a1t = _mm_nt(u0, vt0, k1)
    deg1 = _rowsum(a1, c)
    xp = jnp.pad(x, ((0, P0 - n0), (0, 0)))
    gate0 = jnp.pad(jnp.tanh(vals0), (0, P1 - k1))
    xg1 = _sc_gather(xp, _pad_idx(perm0, P1, P0 - 1), P1) * gate0[:, None]
    h1 = _agg(a1, _zscale(xg1, Wd0, deg1), deg1, bd0, k1)

    score1 = (h1[:k1] @ p1) / jnp.linalg.norm(p1)
    sel1, rank1, perm1, vals1 = _select(score1, k2)
    u1, vt1 = _gather_factors(a1, a1t, perm1, P2)
    a2, a2t = _mm_nt(u1, vt1, k2)
    deg2 = _rowsum(a2, c)
    gate1 = jnp.pad(jnp.tanh(vals1), (0, P2 - k2))
    xg2 = _sc_gather(h1, _pad_idx(perm1, P2, P1 - 1), P2) * gate1[:, None]
    h2 = _agg(a2, _zscale(xg2, Wd1, deg2), deg2, bd1, k2)

    score2 = (h2[:k2] @ p2) / jnp.linalg.norm(p2)
    sel2, rank2, perm2, vals2 = _select(score2, k3)
    u2, vt2 = _gather_factors(a2, a2t, perm2, P3)
    a3, _ = _mm_nt(u2, vt2, k3)
    deg3 = _rowsum(a3, c)
    gate2 = jnp.pad(jnp.tanh(vals2), (0, P3 - k3))
    xg3 = _sc_gather(h2, _pad_idx(perm2, P3, P2 - 1), P3) * gate2[:, None]
    h3 = _agg(a3, _zscale(xg3, Wd2, deg3), deg3, bd2, k3)

    r2 = _unpool(h2, h3, sel2, rank2, P2)
    g2 = _agg(a2, _zscale(r2, Wu0, deg2), deg2, bu0, k2)

    r1 = _unpool(h1, g2, sel1, rank1, P1)
    g1 = _agg(a1, _zscale(r1, Wu1, deg1), deg1, bu1, k1)

    return _unpool(x, g1, sel0, rank0, P0)[:n0]


# speedup vs baseline: 2.3158x; 2.1427x over previous
import functools
import math

import jax
import jax.numpy as jnp
from jax import lax
from jax.experimental import pallas as pl
from jax.experimental.pallas import tpu as pltpu
from jax.experimental.pallas import tpu_sc as plsc


def _rup(n, m):
    return ((n + m - 1) // m) * m


def _blk(p):
    return 512 if p % 512 == 0 else 256


def _mm_nt_body(u_ref, v_ref, o_ref, ot_ref, acc_ref, *, ksteps, n_real, bm,
                bn, want_t):
    @pl.when(pl.program_id(2) == 0)
    def _():
        acc_ref[...] = jnp.zeros_like(acc_ref)

    acc_ref[...] += jax.lax.dot_general(
        u_ref[...], v_ref[...], (((1,), (1,)), ((), ())),
        preferred_element_type=jnp.float32)

    @pl.when(pl.program_id(2) == ksteps - 1)
    def _():
        mi = pl.program_id(0)
        nj = pl.program_id(1)
        rows = mi * bm + jax.lax.broadcasted_iota(jnp.int32, (bm, bn), 0)
        cols = nj * bn + jax.lax.broadcasted_iota(jnp.int32, (bm, bn), 1)
        valid = (rows < n_real) & (cols < n_real)
        res = jnp.where(valid & (rows != cols), acc_ref[...], 0.0)
        res = res + jnp.where(valid & (rows == cols), 1.0, 0.0)
        o_ref[...] = res
        ot_ref[...] = res.T


def _mm_nt(u, vt, n_real):
    u = u.astype(jnp.bfloat16)
    vt = vt.astype(jnp.bfloat16)
    m, k = u.shape
    n = vt.shape[0]
    bm, bn, bk = _blk(m), _blk(n), _blk(k)
    grid = (m // bm, n // bn, k // bk)
    return pl.pallas_call(
        functools.partial(_mm_nt_body, ksteps=grid[2], n_real=n_real,
                          bm=bm, bn=bn, want_t=True),
        grid=grid,
        in_specs=[
            pl.BlockSpec((bm, bk), lambda i, j, kk: (i, kk)),
            pl.BlockSpec((bn, bk), lambda i, j, kk: (j, kk)),
        ],
        out_specs=[pl.BlockSpec((bm, bn), lambda i, j, kk: (i, j)),
                   pl.BlockSpec((bn, bm), lambda i, j, kk: (j, i))],
        out_shape=[jax.ShapeDtypeStruct((m, n), jnp.float32),
                   jax.ShapeDtypeStruct((n, m), jnp.float32)],
        scratch_shapes=[pltpu.VMEM((bm, bn), jnp.float32)],
        compiler_params=pltpu.CompilerParams(
            dimension_semantics=("parallel", "parallel", "arbitrary")),
    )(u, vt)


def _rowsum_body(a_ref, o_ref, *, nsteps, c):
    @pl.when(pl.program_id(1) == 0)
    def _():
        o_ref[...] = jnp.zeros_like(o_ref)

    s = jnp.sum(a_ref[...], axis=1, keepdims=True)
    o_ref[...] += jnp.broadcast_to(s, o_ref.shape)

    @pl.when(pl.program_id(1) == nsteps - 1)
    def _():
        o_ref[...] += 1.0


def _rowsum(a, c):
    m, n = a.shape
    bm, bn = _blk(m), _blk(n)
    grid = (m // bm, n // bn)
    return pl.pallas_call(
        functools.partial(_rowsum_body, nsteps=grid[1], c=c),
        grid=grid,
        in_specs=[pl.BlockSpec((bm, bn), lambda i, j: (i, j))],
        out_specs=pl.BlockSpec((bm, c), lambda i, j: (i, 0)),
        out_shape=jax.ShapeDtypeStruct((m, c), jnp.float32),
        compiler_params=pltpu.CompilerParams(
            dimension_semantics=("parallel", "arbitrary")),
    )(a)


def _zscale_body(inp_ref, w_ref, deg_ref, o_ref):
    xw = jnp.dot(inp_ref[...], w_ref[...], preferred_element_type=jnp.float32)
    o_ref[...] = xw * jax.lax.rsqrt(deg_ref[...])


def _zscale(inp, w, deg):
    m, c = inp.shape
    bm = _blk(m)
    grid = (m // bm,)
    return pl.pallas_call(
        _zscale_body,
        grid=grid,
        in_specs=[
            pl.BlockSpec((bm, c), lambda i: (i, 0)),
            pl.BlockSpec((c, c), lambda i: (0, 0)),
            pl.BlockSpec((bm, c), lambda i: (i, 0)),
        ],
        out_specs=pl.BlockSpec((bm, c), lambda i: (i, 0)),
        out_shape=jax.ShapeDtypeStruct((m, c), jnp.float32),
    )(inp, w, deg)


def _agg_body(a_ref, z_ref, zr_ref, deg_ref, b_ref, o_ref, acc_ref, *,
              ksteps, n_real, bm):
    @pl.when(pl.program_id(1) == 0)
    def _():
        acc_ref[...] = jnp.zeros_like(acc_ref)

    acc_ref[...] += jnp.dot(a_ref[...], z_ref[...],
                            preferred_element_type=jnp.float32)

    @pl.when(pl.program_id(1) == ksteps - 1)
    def _():
        dinv = jax.lax.rsqrt(deg_ref[...])
        h = dinv * (acc_ref[...] + zr_ref[...]) + b_ref[...]
        rows = pl.program_id(0) * bm + jax.lax.broadcasted_iota(
            jnp.int32, h.shape, 0)
        o_ref[...] = jnp.where(rows < n_real, jnp.maximum(h, 0.0), 0.0)


def _agg(a, z, deg, b, n_real):
    m, k = a.shape
    c = z.shape[1]
    bm, bk = _blk(m), _blk(k)
    grid = (m // bm, k // bk)
    return pl.pallas_call(
        functools.partial(_agg_body, ksteps=grid[1], n_real=n_real, bm=bm),
        grid=grid,
        in_specs=[
            pl.BlockSpec((bm, bk), lambda i, kk: (i, kk)),
            pl.BlockSpec((bk, c), lambda i, kk: (kk, 0)),
            pl.BlockSpec((bm, c), lambda i, kk: (i, 0)),
            pl.BlockSpec((bm, c), lambda i, kk: (i, 0)),
            pl.BlockSpec((1, c), lambda i, kk: (0, 0)),
        ],
        out_specs=pl.BlockSpec((bm, c), lambda i, kk: (i, 0)),
        out_shape=jax.ShapeDtypeStruct((m, c), jnp.float32),
        scratch_shapes=[pltpu.VMEM((bm, c), jnp.float32)],
        compiler_params=pltpu.CompilerParams(
            dimension_semantics=("parallel", "arbitrary")),
    )(a, z, z, deg, b.reshape(1, c))


_NBUF = 12
_IDXW = 128


def _sc_build_factors(src, dst, rank_tbl, kp, np_):
    e_tot = src.shape[0]
    n_tiles = 16
    eps = e_tot // n_tiles
    rows_dma = eps // _IDXW
    assert eps % (_IDXW * _NBUF) == 0
    groups = rows_dma // _NBUF
    rows_sc = kp // 2
    brows = 128
    while rows_sc % brows:
        brows -= 32
    nblocks = rows_sc // brows
    dump = brows * np_
    sh_words = dump + 32768
    stripe = (brows * np_) // n_tiles
    zch = stripe
    nzc = 1
    while zch > 12288:
        nzc *= 2
        zch = stripe // nzc
    assert stripe == zch * nzc
    tblsz = rank_tbl.shape[0]
    mesh = plsc.VectorSubcoreMesh(core_axis_name="c", subcore_axis_name="s")

    @functools.partial(
        pl.kernel, mesh=mesh,
        out_type=[jax.ShapeDtypeStruct((kp * np_,), jnp.float32),
                  jax.ShapeDtypeStruct((kp * np_,), jnp.float32)],
        scratch_types=[
            pltpu.VMEM((eps,), jnp.int32),
            pltpu.VMEM((eps,), jnp.int32),
            pltpu.VMEM((tblsz,), jnp.int32),
            pltpu.VMEM((_NBUF, _IDXW), jnp.int32),
            pltpu.VMEM((_IDXW,), jnp.float32),
            pltpu.VMEM((zch,), jnp.float32),
            pltpu.VMEM_SHARED((sh_words,), jnp.float32),
            pltpu.SemaphoreType.DMA,
        ],
        compiler_params=pltpu.CompilerParams(needs_layout_passes=False),
    )
    def k(src_hbm, dst_hbm, rank_hbm, u_hbm, vt_hbm, src_v, dst_v, rank_v,
          idx_v, val_v, zero_v, shared, sem):
        sc = lax.axis_index("c")
        tid = lax.axis_index("s")
        pltpu.sync_copy(src_hbm.at[pl.ds(tid * eps, eps)], src_v)
        pltpu.sync_copy(dst_hbm.at[pl.ds(tid * eps, eps)], dst_v)
        pltpu.sync_copy(rank_hbm, rank_v)

        def fillz(i, _):
            zero_v[pl.ds(i * 16, 16)] = jnp.zeros((16,), jnp.float32)
            return 0

        lax.fori_loop(0, zch // 16, fillz, 0)

        def fill1(i, _):
            val_v[pl.ds(i * 16, 16)] = jnp.ones((16,), jnp.float32)
            return 0

        lax.fori_loop(0, _IDXW // 16, fill1, 0)

        for out_hbm, key_v, col_v in ((u_hbm, src_v, dst_v),
                                      (vt_hbm, dst_v, src_v)):
            def block(b, _):
                r0 = sc * rows_sc + b * brows

                def zero(i, _):
                    pltpu.sync_copy(
                        zero_v,
                        shared.at[pl.ds(tid * stripe + i * zch, zch)])
                    return 0

                lax.fori_loop(0, nzc, zero, 0)
                plsc.subcore_barrier()

                def group(g, _):
                    handles = []
                    for j in range(_NBUF):
                        def chunk(ci, _, j=j, g=g):
                            off = (g * _NBUF + j) * _IDXW + ci * 16
                            keys = key_v[pl.ds(off, 16)]
                            cols = col_v[pl.ds(off, 16)]
                            rl = plsc.load_gather(rank_v, [keys]) - r0
                            inb = (rl >= 0) & (rl < brows)
                            idx_v[j, pl.ds(ci * 16, 16)] = jnp.where(
                                inb, rl * np_ + cols, dump + (keys & 32767))
                            return 0

                        lax.fori_loop(0, _IDXW // 16, chunk, 0)
                        handles.append(pltpu.async_copy(
                            val_v, shared.at[idx_v.at[j]], sem, add=True))
                    for h in handles:
                        h.wait()
                    return 0

                lax.fori_loop(0, groups, group, 0)
                plsc.subcore_barrier()
                pltpu.sync_copy(
                    shared.at[pl.ds(tid * stripe, stripe)],
                    out_hbm.at[pl.ds(r0 * np_ + tid * stripe, stripe)])
                plsc.subcore_barrier()
                return 0

            lax.fori_loop(0, nblocks, block, 0)

    return k(src, dst, rank_tbl)


def _sc_gather(table, idx, out_rows):
    t_rows, d = table.shape
    nw = 32
    rpw = out_rows // nw
    assert out_rows % nw == 0
    cap = max(8, (384 * 1024) // (d * 4) // 8 * 8)
    c_rows = min(rpw, cap)
    while rpw % c_rows:
        c_rows -= 8
    nchunks = rpw // c_rows
    idx_buf = max(16, c_rows)
    mesh = plsc.VectorSubcoreMesh(core_axis_name="c", subcore_axis_name="s")

    @functools.partial(
        pl.kernel, mesh=mesh,
        out_type=jax.ShapeDtypeStruct((out_rows, d), jnp.float32),
        scratch_types=[
            pltpu.VMEM((idx_buf,), jnp.int32),
            pltpu.VMEM((c_rows, d), jnp.float32),
            pltpu.SemaphoreType.DMA,
        ],
    )
    def k(table_hbm, idx_hbm, out_hbm, idx_v, rows_v, sem):
        wid = lax.axis_index("s") * 2 + lax.axis_index("c")
        for j in range(nchunks):
            base = wid * rpw + j * c_rows
            pltpu.sync_copy(idx_hbm.at[pl.ds(base, c_rows)],
                            idx_v.at[pl.ds(0, c_rows)])
            pltpu.async_copy(table_hbm.at[idx_v.at[pl.ds(0, c_rows)]],
                             rows_v, sem).wait()
            pltpu.sync_copy(rows_v, out_hbm.at[pl.ds(base, c_rows)])

    return k(table, idx)


def _pad_idx(perm, out_rows, zero_row):
    k = perm.shape[0]
    return jnp.concatenate(
        [perm, jnp.full((out_rows - k,), zero_row, jnp.int32)])


def _select(score, k):
    vals = jax.lax.top_k(score, k)[0]
    t = vals[k - 1]
    gt = score > t
    ngt = jnp.sum(gt.astype(jnp.int32))
    eq = score == t
    tie = eq & (jnp.cumsum(eq.astype(jnp.int32)) <= (k - ngt))
    sel = gt | tie
    rank = (jnp.cumsum(sel.astype(jnp.int32)) - 1).astype(jnp.int32)
    perm = jnp.nonzero(sel, size=k)[0].astype(jnp.int32)
    return sel, rank, perm, score[perm]


def _pad_rows(a, p):
    return jnp.pad(a, ((0, p - a.shape[0]), (0, 0)))


def _gather_factors(am, amt, perm, kp):
    k = perm.shape[0]
    psrc = am.shape[0]
    idx = _pad_idx(perm, kp, psrc - 1)
    u = _sc_gather(am, idx, kp)
    vt = _sc_gather(amt, idx, kp)
    return u, vt


def _unpool(res, hnext, sel, rank, out_rows):
    pn = hnext.shape[0]
    idx = jnp.where(sel, rank, pn - 1).astype(jnp.int32)
    idx = _pad_idx(idx, out_rows, pn - 1)
    up = _sc_gather(hnext, idx, out_rows)
    return res + up[: res.shape[0]]


def kernel(x, edge_index, p0, p1, p2, Wd0, Wd1, Wd2, bd0, bd1, bd2,
           Wu0, Wu1, bu0, bu1):
    n0, c = x.shape
    k1 = -(-n0 // 2)
    k2 = -(-k1 // 2)
    k3 = -(-k2 // 2)
    P0, P1, P2, P3 = (_rup(v, 256) for v in (n0, k1, k2, k3))

    src, dst = edge_index[0], edge_index[1]

    score0 = (x @ p0) / jnp.linalg.norm(p0)
    sel0, rank0, perm0, vals0 = _select(score0, k1)

    e_pad = _rup(edge_index.shape[1] + n0, 16 * 128 * 12)
    loops = jnp.arange(n0, dtype=jnp.int32)
    fill = jnp.full((e_pad - edge_index.shape[1] - n0,), n0, jnp.int32)
    srcp = jnp.concatenate([src, loops, fill])
    dstp = jnp.concatenate([dst, loops, fill])
    rank_tbl = jnp.where(sel0, rank0, -1).astype(jnp.int32)
    rank_tbl = jnp.pad(rank_tbl, (0, P0 - n0), constant_values=-1)
    u0f, vt0f = _sc_build_factors(srcp, dstp, rank_tbl, P1, P0)
    u0 = u0f.reshape(P1, P0)
    vt0 = vt0f.reshape(P1, P0)

    a1, a1t = _mm_nt(u0, vt0, k1)
    deg1 = _rowsum(a1, c)
    xp = jnp.pad(x, ((0, P0 - n0), (0, 0)))
    gate0 = jnp.pad(jnp.tanh(vals0), (0, P1 - k1))
    xg1 = _sc_gather(xp, _pad_idx(perm0, P1, P0 - 1), P1) * gate0[:, None]
    h1 = _agg(a1, _zscale(xg1, Wd0, deg1), deg1, bd0, k1)

    score1 = (h1[:k1] @ p1) / jnp.linalg.norm(p1)
    sel1, rank1, perm1, vals1 = _select(score1, k2)
    u1, vt1 = _gather_factors(a1, a1t, perm1, P2)
    a2, a2t = _mm_nt(u1, vt1, k2)
    deg2 = _rowsum(a2, c)
    gate1 = jnp.pad(jnp.tanh(vals1), (0, P2 - k2))
    xg2 = _sc_gather(h1, _pad_idx(perm1, P2, P1 - 1), P2) * gate1[:, None]
    h2 = _agg(a2, _zscale(xg2, Wd1, deg2), deg2, bd1, k2)

    score2 = (h2[:k2] @ p2) / jnp.linalg.norm(p2)
    sel2, rank2, perm2, vals2 = _select(score2, k3)
    u2, vt2 = _gather_factors(a2, a2t, perm2, P3)
    a3, _ = _mm_nt(u2, vt2, k3)
    deg3 = _rowsum(a3, c)
    gate2 = jnp.pad(jnp.tanh(vals2), (0, P3 - k3))
    xg3 = _sc_gather(h2, _pad_idx(perm2, P3, P2 - 1), P3) * gate2[:, None]
    h3 = _agg(a3, _zscale(xg3, Wd2, deg3), deg3, bd2, k3)

    r2 = _unpool(h2, h3, sel2, rank2, P2)
    g2 = _agg(a2, _zscale(r2, Wu0, deg2), deg2, bu0, k2)

    r1 = _unpool(h1, g2, sel1, rank1, P1)
    g1 = _agg(a1, _zscale(r1, Wu1, deg1), deg1, bu1, k1)

    return _unpool(x, g1, sel0, rank0, P0)[:n0]


# speedup vs baseline: 2.6805x; 1.1575x over previous
import functools
import math

import jax
import jax.numpy as jnp
from jax import lax
from jax.experimental import pallas as pl
from jax.experimental.pallas import tpu as pltpu
from jax.experimental.pallas import tpu_sc as plsc


def _rup(n, m):
    return ((n + m - 1) // m) * m


def _blk(p):
    return 512 if p % 512 == 0 else 256


def _mm_nt_body(u_ref, v_ref, o_ref, ot_ref, acc_ref, *, ksteps, n_real, bm,
                bn, want_t):
    @pl.when(pl.program_id(2) == 0)
    def _():
        acc_ref[...] = jnp.zeros_like(acc_ref)

    acc_ref[...] += jax.lax.dot_general(
        u_ref[...], v_ref[...], (((1,), (1,)), ((), ())),
        preferred_element_type=jnp.float32)

    @pl.when(pl.program_id(2) == ksteps - 1)
    def _():
        mi = pl.program_id(0)
        nj = pl.program_id(1)
        rows = mi * bm + jax.lax.broadcasted_iota(jnp.int32, (bm, bn), 0)
        cols = nj * bn + jax.lax.broadcasted_iota(jnp.int32, (bm, bn), 1)
        valid = (rows < n_real) & (cols < n_real)
        res = jnp.where(valid & (rows != cols), acc_ref[...], 0.0)
        res = res + jnp.where(valid & (rows == cols), 1.0, 0.0)
        o_ref[...] = res
        ot_ref[...] = res.T


def _mm_nt(u, vt, n_real):
    u = u.astype(jnp.bfloat16)
    vt = vt.astype(jnp.bfloat16)
    m, k = u.shape
    n = vt.shape[0]
    bm, bn = _blk(m), _blk(n)
    bk = 1024 if k % 1024 == 0 else _blk(k)
    grid = (m // bm, n // bn, k // bk)
    return pl.pallas_call(
        functools.partial(_mm_nt_body, ksteps=grid[2], n_real=n_real,
                          bm=bm, bn=bn, want_t=True),
        grid=grid,
        in_specs=[
            pl.BlockSpec((bm, bk), lambda i, j, kk: (i, kk)),
            pl.BlockSpec((bn, bk), lambda i, j, kk: (j, kk)),
        ],
        out_specs=[pl.BlockSpec((bm, bn), lambda i, j, kk: (i, j)),
                   pl.BlockSpec((bn, bm), lambda i, j, kk: (j, i))],
        out_shape=[jax.ShapeDtypeStruct((m, n), jnp.float32),
                   jax.ShapeDtypeStruct((n, m), jnp.float32)],
        scratch_shapes=[pltpu.VMEM((bm, bn), jnp.float32)],
        compiler_params=pltpu.CompilerParams(
            dimension_semantics=("parallel", "parallel", "arbitrary")),
    )(u, vt)


def _rowsum_body(a_ref, o_ref, *, nsteps, c):
    @pl.when(pl.program_id(1) == 0)
    def _():
        o_ref[...] = jnp.zeros_like(o_ref)

    s = jnp.sum(a_ref[...], axis=1, keepdims=True)
    o_ref[...] += jnp.broadcast_to(s, o_ref.shape)

    @pl.when(pl.program_id(1) == nsteps - 1)
    def _():
        o_ref[...] += 1.0


def _rowsum(a, c):
    m, n = a.shape
    bm, bn = _blk(m), _blk(n)
    grid = (m // bm, n // bn)
    return pl.pallas_call(
        functools.partial(_rowsum_body, nsteps=grid[1], c=c),
        grid=grid,
        in_specs=[pl.BlockSpec((bm, bn), lambda i, j: (i, j))],
        out_specs=pl.BlockSpec((bm, c), lambda i, j: (i, 0)),
        out_shape=jax.ShapeDtypeStruct((m, c), jnp.float32),
        compiler_params=pltpu.CompilerParams(
            dimension_semantics=("parallel", "arbitrary")),
    )(a)


def _zscale_body(inp_ref, w_ref, deg_ref, o_ref):
    xw = jnp.dot(inp_ref[...], w_ref[...], preferred_element_type=jnp.float32)
    o_ref[...] = xw * jax.lax.rsqrt(deg_ref[...])


def _zscale(inp, w, deg):
    m, c = inp.shape
    bm = _blk(m)
    grid = (m // bm,)
    return pl.pallas_call(
        _zscale_body,
        grid=grid,
        in_specs=[
            pl.BlockSpec((bm, c), lambda i: (i, 0)),
            pl.BlockSpec((c, c), lambda i: (0, 0)),
            pl.BlockSpec((bm, c), lambda i: (i, 0)),
        ],
        out_specs=pl.BlockSpec((bm, c), lambda i: (i, 0)),
        out_shape=jax.ShapeDtypeStruct((m, c), jnp.float32),
    )(inp, w, deg)


def _agg_body(a_ref, z_ref, zr_ref, deg_ref, b_ref, o_ref, acc_ref, *,
              ksteps, n_real, bm):
    @pl.when(pl.program_id(1) == 0)
    def _():
        acc_ref[...] = jnp.zeros_like(acc_ref)

    acc_ref[...] += jnp.dot(a_ref[...], z_ref[...],
                            preferred_element_type=jnp.float32)

    @pl.when(pl.program_id(1) == ksteps - 1)
    def _():
        dinv = jax.lax.rsqrt(deg_ref[...])
        h = dinv * (acc_ref[...] + zr_ref[...]) + b_ref[...]
        rows = pl.program_id(0) * bm + jax.lax.broadcasted_iota(
            jnp.int32, h.shape, 0)
        o_ref[...] = jnp.where(rows < n_real, jnp.maximum(h, 0.0), 0.0)


def _agg(a, z, deg, b, n_real):
    m, k = a.shape
    c = z.shape[1]
    bm, bk = _blk(m), _blk(k)
    grid = (m // bm, k // bk)
    return pl.pallas_call(
        functools.partial(_agg_body, ksteps=grid[1], n_real=n_real, bm=bm),
        grid=grid,
        in_specs=[
            pl.BlockSpec((bm, bk), lambda i, kk: (i, kk)),
            pl.BlockSpec((bk, c), lambda i, kk: (kk, 0)),
            pl.BlockSpec((bm, c), lambda i, kk: (i, 0)),
            pl.BlockSpec((bm, c), lambda i, kk: (i, 0)),
            pl.BlockSpec((1, c), lambda i, kk: (0, 0)),
        ],
        out_specs=pl.BlockSpec((bm, c), lambda i, kk: (i, 0)),
        out_shape=jax.ShapeDtypeStruct((m, c), jnp.float32),
        scratch_shapes=[pltpu.VMEM((bm, c), jnp.float32)],
        compiler_params=pltpu.CompilerParams(
            dimension_semantics=("parallel", "arbitrary")),
    )(a, z, z, deg, b.reshape(1, c))


_NBUF = 12
_IDXW = 128


def _sc_build_factors(src, dst, rank_tbl, kp, np_):
    e_tot = src.shape[0]
    n_tiles = 16
    eps = e_tot // n_tiles
    rows_dma = eps // _IDXW
    assert eps % (_IDXW * _NBUF) == 0
    groups = rows_dma // _NBUF
    rows_sc = kp // 2
    brows = 128
    while rows_sc % brows:
        brows -= 32
    nblocks = rows_sc // brows
    dump = brows * np_
    sh_words = dump + 32768
    stripe = (brows * np_) // n_tiles
    zch = stripe
    nzc = 1
    while zch > 12288:
        nzc *= 2
        zch = stripe // nzc
    assert stripe == zch * nzc
    tblsz = rank_tbl.shape[0]
    mesh = plsc.VectorSubcoreMesh(core_axis_name="c", subcore_axis_name="s")

    @functools.partial(
        pl.kernel, mesh=mesh,
        out_type=[jax.ShapeDtypeStruct((kp * np_,), jnp.float32),
                  jax.ShapeDtypeStruct((kp * np_,), jnp.float32)],
        scratch_types=[
            pltpu.VMEM((eps,), jnp.int32),
            pltpu.VMEM((eps,), jnp.int32),
            pltpu.VMEM((tblsz,), jnp.int32),
            pltpu.VMEM((_NBUF, _IDXW), jnp.int32),
            pltpu.VMEM((_IDXW,), jnp.float32),
            pltpu.VMEM((zch,), jnp.float32),
            pltpu.VMEM_SHARED((sh_words,), jnp.float32),
            pltpu.SemaphoreType.DMA,
        ],
        compiler_params=pltpu.CompilerParams(needs_layout_passes=False),
    )
    def k(src_hbm, dst_hbm, rank_hbm, u_hbm, vt_hbm, src_v, dst_v, rank_v,
          idx_v, val_v, zero_v, shared, sem):
        sc = lax.axis_index("c")
        tid = lax.axis_index("s")
        pltpu.sync_copy(src_hbm.at[pl.ds(tid * eps, eps)], src_v)
        pltpu.sync_copy(dst_hbm.at[pl.ds(tid * eps, eps)], dst_v)
        pltpu.sync_copy(rank_hbm, rank_v)

        def fillz(i, _):
            zero_v[pl.ds(i * 16, 16)] = jnp.zeros((16,), jnp.float32)
            return 0

        lax.fori_loop(0, zch // 16, fillz, 0)

        def fill1(i, _):
            val_v[pl.ds(i * 16, 16)] = jnp.ones((16,), jnp.float32)
            return 0

        lax.fori_loop(0, _IDXW // 16, fill1, 0)

        for out_hbm, key_v, col_v in ((u_hbm, src_v, dst_v),
                                      (vt_hbm, dst_v, src_v)):
            def block(b, _):
                r0 = sc * rows_sc + b * brows

                def zero(i, _):
                    pltpu.sync_copy(
                        zero_v,
                        shared.at[pl.ds(tid * stripe + i * zch, zch)])
                    return 0

                lax.fori_loop(0, nzc, zero, 0)
                plsc.subcore_barrier()

                def group(g, _):
                    handles = []
                    for j in range(_NBUF):
                        def chunk(ci, _, j=j, g=g):
                            off = (g * _NBUF + j) * _IDXW + ci * 16
                            keys = key_v[pl.ds(off, 16)]
                            cols = col_v[pl.ds(off, 16)]
                            rl = plsc.load_gather(rank_v, [keys]) - r0
                            inb = (rl >= 0) & (rl < brows)
                            idx_v[j, pl.ds(ci * 16, 16)] = jnp.where(
                                inb, rl * np_ + cols, dump + (keys & 32767))
                            return 0

                        lax.fori_loop(0, _IDXW // 16, chunk, 0)
                        handles.append(pltpu.async_copy(
                            val_v, shared.at[idx_v.at[j]], sem, add=True))
                    for h in handles:
                        h.wait()
                    return 0

                lax.fori_loop(0, groups, group, 0)
                plsc.subcore_barrier()
                pltpu.sync_copy(
                    shared.at[pl.ds(tid * stripe, stripe)],
                    out_hbm.at[pl.ds(r0 * np_ + tid * stripe, stripe)])
                plsc.subcore_barrier()
                return 0

            lax.fori_loop(0, nblocks, block, 0)

    return k(src, dst, rank_tbl)


def _sc_gather(table, idx, out_rows):
    t_rows, d = table.shape
    nw = 32
    rpw = out_rows // nw
    assert out_rows % nw == 0
    cap = max(8, (384 * 1024) // (d * 4) // 8 * 8)
    c_rows = min(rpw, cap)
    while rpw % c_rows:
        c_rows -= 8
    nchunks = rpw // c_rows
    idx_buf = max(16, c_rows)
    mesh = plsc.VectorSubcoreMesh(core_axis_name="c", subcore_axis_name="s")

    @functools.partial(
        pl.kernel, mesh=mesh,
        out_type=jax.ShapeDtypeStruct((out_rows, d), jnp.float32),
        scratch_types=[
            pltpu.VMEM((idx_buf,), jnp.int32),
            pltpu.VMEM((c_rows, d), jnp.float32),
            pltpu.SemaphoreType.DMA,
        ],
    )
    def k(table_hbm, idx_hbm, out_hbm, idx_v, rows_v, sem):
        wid = lax.axis_index("s") * 2 + lax.axis_index("c")
        for j in range(nchunks):
            base = wid * rpw + j * c_rows
            pltpu.sync_copy(idx_hbm.at[pl.ds(base, c_rows)],
                            idx_v.at[pl.ds(0, c_rows)])
            pltpu.async_copy(table_hbm.at[idx_v.at[pl.ds(0, c_rows)]],
                             rows_v, sem).wait()
            pltpu.sync_copy(rows_v, out_hbm.at[pl.ds(base, c_rows)])

    return k(table, idx)


def _pad_idx(perm, out_rows, zero_row):
    k = perm.shape[0]
    return jnp.concatenate(
        [perm, jnp.full((out_rows - k,), zero_row, jnp.int32)])


def _select(score, k):
    vals = jax.lax.top_k(score, k)[0]
    t = vals[k - 1]
    gt = score > t
    ngt = jnp.sum(gt.astype(jnp.int32))
    eq = score == t
    tie = eq & (jnp.cumsum(eq.astype(jnp.int32)) <= (k - ngt))
    sel = gt | tie
    rank = (jnp.cumsum(sel.astype(jnp.int32)) - 1).astype(jnp.int32)
    perm = jnp.nonzero(sel, size=k)[0].astype(jnp.int32)
    return sel, rank, perm, score[perm]


def _pad_rows(a, p):
    return jnp.pad(a, ((0, p - a.shape[0]), (0, 0)))


def _gather_factors(am, amt, perm, kp):
    k = perm.shape[0]
    psrc = am.shape[0]
    idx = _pad_idx(perm, kp, psrc - 1)
    u = _sc_gather(am, idx, kp)
    vt = _sc_gather(amt, idx, kp)
    return u, vt


def _unpool(res, hnext, sel, rank, out_rows):
    pn = hnext.shape[0]
    idx = jnp.where(sel, rank, pn - 1).astype(jnp.int32)
    idx = _pad_idx(idx, out_rows, pn - 1)
    up = _sc_gather(hnext, idx, out_rows)
    return res + up[: res.shape[0]]


def kernel(x, edge_index, p0, p1, p2, Wd0, Wd1, Wd2, bd0, bd1, bd2,
           Wu0, Wu1, bu0, bu1):
    n0, c = x.shape
    k1 = -(-n0 // 2)
    k2 = -(-k1 // 2)
    k3 = -(-k2 // 2)
    P0, P1, P2, P3 = (_rup(v, 256) for v in (n0, k1, k2, k3))

    src, dst = edge_index[0], edge_index[1]

    score0 = (x @ p0) / jnp.linalg.norm(p0)
    sel0, rank0, perm0, vals0 = _select(score0, k1)

    e_pad = _rup(edge_index.shape[1] + n0, 16 * 128 * 12)
    loops = jnp.arange(n0, dtype=jnp.int32)
    fill = jnp.full((e_pad - edge_index.shape[1] - n0,), n0, jnp.int32)
    srcp = jnp.concatenate([src, loops, fill])
    dstp = jnp.concatenate([dst, loops, fill])
    rank_tbl = jnp.where(sel0, rank0, -1).astype(jnp.int32)
    rank_tbl = jnp.pad(rank_tbl, (0, P0 - n0), constant_values=-1)
    u0f, vt0f = _sc_build_factors(srcp, dstp, rank_tbl, P1, P0)
    u0 = u0f.reshape(P1, P0)
    vt0 = vt0f.reshape(P1, P0)

    a1, a1t = _mm_nt(u0, vt0, k1)
    deg1 = _rowsum(a1, c)
    xp = jnp.pad(x, ((0, P0 - n0), (0, 0)))
    gate0 = jnp.pad(jnp.tanh(vals0), (0, P1 - k1))
    xg1 = _sc_gather(xp, _pad_idx(perm0, P1, P0 - 1), P1) * gate0[:, None]
    h1 = _agg(a1, _zscale(xg1, Wd0, deg1), deg1, bd0, k1)

    score1 = (h1[:k1] @ p1) / jnp.linalg.norm(p1)
    sel1, rank1, perm1, vals1 = _select(score1, k2)
    u1, vt1 = _gather_factors(a1, a1t, perm1, P2)
    a2, a2t = _mm_nt(u1, vt1, k2)
    deg2 = _rowsum(a2, c)
    gate1 = jnp.pad(jnp.tanh(vals1), (0, P2 - k2))
    xg2 = _sc_gather(h1, _pad_idx(perm1, P2, P1 - 1), P2) * gate1[:, None]
    h2 = _agg(a2, _zscale(xg2, Wd1, deg2), deg2, bd1, k2)

    score2 = (h2[:k2] @ p2) / jnp.linalg.norm(p2)
    sel2, rank2, perm2, vals2 = _select(score2, k3)
    u2, vt2 = _gather_factors(a2, a2t, perm2, P3)
    a3, _ = _mm_nt(u2, vt2, k3)
    deg3 = _rowsum(a3, c)
    gate2 = jnp.pad(jnp.tanh(vals2), (0, P3 - k3))
    xg3 = _sc_gather(h2, _pad_idx(perm2, P3, P2 - 1), P3) * gate2[:, None]
    h3 = _agg(a3, _zscale(xg3, Wd2, deg3), deg3, bd2, k3)

    r2 = _unpool(h2, h3, sel2, rank2, P2)
    g2 = _agg(a2, _zscale(r2, Wu0, deg2), deg2, bu0, k2)

    r1 = _unpool(h1, g2, sel1, rank1, P1)
    g1 = _agg(a1, _zscale(r1, Wu1, deg1), deg1, bu1, k1)

    return _unpool(x, g1, sel0, rank0, P0)[:n0]


# speedup vs baseline: 3.1861x; 1.1886x over previous
import functools
import math

import jax
import jax.numpy as jnp
from jax import lax
from jax.experimental import pallas as pl
from jax.experimental.pallas import tpu as pltpu
from jax.experimental.pallas import tpu_sc as plsc


def _rup(n, m):
    return ((n + m - 1) // m) * m


def _blk(p):
    return 512 if p % 512 == 0 else 256


def _mm_nt_body(u_ref, v_ref, o_ref, ot_ref, acc_ref, *, ksteps, n_real, bm,
                bn, want_t):
    @pl.when(pl.program_id(2) == 0)
    def _():
        acc_ref[...] = jnp.zeros_like(acc_ref)

    acc_ref[...] += jax.lax.dot_general(
        u_ref[...], v_ref[...], (((1,), (1,)), ((), ())),
        preferred_element_type=jnp.float32)

    @pl.when(pl.program_id(2) == ksteps - 1)
    def _():
        mi = pl.program_id(0)
        nj = pl.program_id(1)
        rows = mi * bm + jax.lax.broadcasted_iota(jnp.int32, (bm, bn), 0)
        cols = nj * bn + jax.lax.broadcasted_iota(jnp.int32, (bm, bn), 1)
        valid = (rows < n_real) & (cols < n_real)
        res = jnp.where(valid & (rows != cols), acc_ref[...], 0.0)
        res = res + jnp.where(valid & (rows == cols), 1.0, 0.0)
        o_ref[...] = res
        ot_ref[...] = res.T


def _mm_nt(u, vt, n_real):
    u = u.astype(jnp.bfloat16)
    vt = vt.astype(jnp.bfloat16)
    m, k = u.shape
    n = vt.shape[0]
    bm = 1024 if m % 1024 == 0 else _blk(m)
    bn = 1024 if n % 1024 == 0 else _blk(n)
    bk = 1024 if k % 1024 == 0 else _blk(k)
    grid = (m // bm, n // bn, k // bk)
    return pl.pallas_call(
        functools.partial(_mm_nt_body, ksteps=grid[2], n_real=n_real,
                          bm=bm, bn=bn, want_t=True),
        grid=grid,
        in_specs=[
            pl.BlockSpec((bm, bk), lambda i, j, kk: (i, kk)),
            pl.BlockSpec((bn, bk), lambda i, j, kk: (j, kk)),
        ],
        out_specs=[pl.BlockSpec((bm, bn), lambda i, j, kk: (i, j)),
                   pl.BlockSpec((bn, bm), lambda i, j, kk: (j, i))],
        out_shape=[jax.ShapeDtypeStruct((m, n), jnp.float32),
                   jax.ShapeDtypeStruct((n, m), jnp.float32)],
        scratch_shapes=[pltpu.VMEM((bm, bn), jnp.float32)],
        compiler_params=pltpu.CompilerParams(
            dimension_semantics=("parallel", "parallel", "arbitrary")),
    )(u, vt)


def _rowsum_body(a_ref, o_ref, *, nsteps, c):
    @pl.when(pl.program_id(1) == 0)
    def _():
        o_ref[...] = jnp.zeros_like(o_ref)

    s = jnp.sum(a_ref[...], axis=1, keepdims=True)
    o_ref[...] += jnp.broadcast_to(s, o_ref.shape)

    @pl.when(pl.program_id(1) == nsteps - 1)
    def _():
        o_ref[...] += 1.0


def _rowsum(a, c):
    m, n = a.shape
    bm, bn = _blk(m), _blk(n)
    grid = (m // bm, n // bn)
    return pl.pallas_call(
        functools.partial(_rowsum_body, nsteps=grid[1], c=c),
        grid=grid,
        in_specs=[pl.BlockSpec((bm, bn), lambda i, j: (i, j))],
        out_specs=pl.BlockSpec((bm, c), lambda i, j: (i, 0)),
        out_shape=jax.ShapeDtypeStruct((m, c), jnp.float32),
        compiler_params=pltpu.CompilerParams(
            dimension_semantics=("parallel", "arbitrary")),
    )(a)


def _zscale_body(inp_ref, w_ref, deg_ref, o_ref):
    xw = jnp.dot(inp_ref[...], w_ref[...], preferred_element_type=jnp.float32)
    o_ref[...] = xw * jax.lax.rsqrt(deg_ref[...])


def _zscale(inp, w, deg):
    m, c = inp.shape
    bm = _blk(m)
    grid = (m // bm,)
    return pl.pallas_call(
        _zscale_body,
        grid=grid,
        in_specs=[
            pl.BlockSpec((bm, c), lambda i: (i, 0)),
            pl.BlockSpec((c, c), lambda i: (0, 0)),
            pl.BlockSpec((bm, c), lambda i: (i, 0)),
        ],
        out_specs=pl.BlockSpec((bm, c), lambda i: (i, 0)),
        out_shape=jax.ShapeDtypeStruct((m, c), jnp.float32),
    )(inp, w, deg)


def _agg_body(a_ref, z_ref, zr_ref, deg_ref, b_ref, o_ref, acc_ref, *,
              ksteps, n_real, bm):
    @pl.when(pl.program_id(1) == 0)
    def _():
        acc_ref[...] = jnp.zeros_like(acc_ref)

    acc_ref[...] += jnp.dot(a_ref[...], z_ref[...],
                            preferred_element_type=jnp.float32)

    @pl.when(pl.program_id(1) == ksteps - 1)
    def _():
        dinv = jax.lax.rsqrt(deg_ref[...])
        h = dinv * (acc_ref[...] + zr_ref[...]) + b_ref[...]
        rows = pl.program_id(0) * bm + jax.lax.broadcasted_iota(
            jnp.int32, h.shape, 0)
        o_ref[...] = jnp.where(rows < n_real, jnp.maximum(h, 0.0), 0.0)


def _agg(a, z, deg, b, n_real):
    m, k = a.shape
    c = z.shape[1]
    bm, bk = _blk(m), _blk(k)
    grid = (m // bm, k // bk)
    return pl.pallas_call(
        functools.partial(_agg_body, ksteps=grid[1], n_real=n_real, bm=bm),
        grid=grid,
        in_specs=[
            pl.BlockSpec((bm, bk), lambda i, kk: (i, kk)),
            pl.BlockSpec((bk, c), lambda i, kk: (kk, 0)),
            pl.BlockSpec((bm, c), lambda i, kk: (i, 0)),
            pl.BlockSpec((bm, c), lambda i, kk: (i, 0)),
            pl.BlockSpec((1, c), lambda i, kk: (0, 0)),
        ],
        out_specs=pl.BlockSpec((bm, c), lambda i, kk: (i, 0)),
        out_shape=jax.ShapeDtypeStruct((m, c), jnp.float32),
        scratch_shapes=[pltpu.VMEM((bm, c), jnp.float32)],
        compiler_params=pltpu.CompilerParams(
            dimension_semantics=("parallel", "arbitrary")),
    )(a, z, z, deg, b.reshape(1, c))


_NBUF = 12
_IDXW = 128


def _sc_build_factors(src, dst, rank_tbl, kp, np_):
    e_tot = src.shape[0]
    n_tiles = 16
    eps = e_tot // n_tiles
    rows_dma = eps // _IDXW
    assert eps % (_IDXW * _NBUF) == 0
    groups = rows_dma // _NBUF
    rows_sc = kp // 2
    brows = 128
    while rows_sc % brows:
        brows -= 32
    nblocks = rows_sc // brows
    dump = brows * np_
    sh_words = dump + 32768
    stripe = (brows * np_) // n_tiles
    zch = stripe
    nzc = 1
    while zch > 12288:
        nzc *= 2
        zch = stripe // nzc
    assert stripe == zch * nzc
    tblsz = rank_tbl.shape[0]
    mesh = plsc.VectorSubcoreMesh(core_axis_name="c", subcore_axis_name="s")

    @functools.partial(
        pl.kernel, mesh=mesh,
        out_type=[jax.ShapeDtypeStruct((kp * np_,), jnp.float32),
                  jax.ShapeDtypeStruct((kp * np_,), jnp.float32)],
        scratch_types=[
            pltpu.VMEM((eps,), jnp.int32),
            pltpu.VMEM((eps,), jnp.int32),
            pltpu.VMEM((tblsz,), jnp.int32),
            pltpu.VMEM((_NBUF, _IDXW), jnp.int32),
            pltpu.VMEM((_IDXW,), jnp.float32),
            pltpu.VMEM((zch,), jnp.float32),
            pltpu.VMEM_SHARED((sh_words,), jnp.float32),
            pltpu.SemaphoreType.DMA,
        ],
        compiler_params=pltpu.CompilerParams(needs_layout_passes=False),
    )
    def k(src_hbm, dst_hbm, rank_hbm, u_hbm, vt_hbm, src_v, dst_v, rank_v,
          idx_v, val_v, zero_v, shared, sem):
        sc = lax.axis_index("c")
        tid = lax.axis_index("s")
        pltpu.sync_copy(src_hbm.at[pl.ds(tid * eps, eps)], src_v)
        pltpu.sync_copy(dst_hbm.at[pl.ds(tid * eps, eps)], dst_v)
        pltpu.sync_copy(rank_hbm, rank_v)

        def fillz(i, _):
            zero_v[pl.ds(i * 16, 16)] = jnp.zeros((16,), jnp.float32)
            return 0

        lax.fori_loop(0, zch // 16, fillz, 0)

        def fill1(i, _):
            val_v[pl.ds(i * 16, 16)] = jnp.ones((16,), jnp.float32)
            return 0

        lax.fori_loop(0, _IDXW // 16, fill1, 0)

        for out_hbm, key_v, col_v in ((u_hbm, src_v, dst_v),
                                      (vt_hbm, dst_v, src_v)):
            def block(b, _):
                r0 = sc * rows_sc + b * brows

                def zero(i, _):
                    pltpu.sync_copy(
                        zero_v,
                        shared.at[pl.ds(tid * stripe + i * zch, zch)])
                    return 0

                lax.fori_loop(0, nzc, zero, 0)
                plsc.subcore_barrier()

                def group(g, _):
                    handles = []
                    for j in range(_NBUF):
                        def chunk(ci, _, j=j, g=g):
                            off = (g * _NBUF + j) * _IDXW + ci * 16
                            keys = key_v[pl.ds(off, 16)]
                            cols = col_v[pl.ds(off, 16)]
                            rl = plsc.load_gather(rank_v, [keys]) - r0
                            inb = (rl >= 0) & (rl < brows)
                            idx_v[j, pl.ds(ci * 16, 16)] = jnp.where(
                                inb, rl * np_ + cols, dump + (keys & 32767))
                            return 0

                        lax.fori_loop(0, _IDXW // 16, chunk, 0)
                        handles.append(pltpu.async_copy(
                            val_v, shared.at[idx_v.at[j]], sem, add=True))
                    for h in handles:
                        h.wait()
                    return 0

                lax.fori_loop(0, groups, group, 0)
                plsc.subcore_barrier()
                pltpu.sync_copy(
                    shared.at[pl.ds(tid * stripe, stripe)],
                    out_hbm.at[pl.ds(r0 * np_ + tid * stripe, stripe)])
                plsc.subcore_barrier()
                return 0

            lax.fori_loop(0, nblocks, block, 0)

    return k(src, dst, rank_tbl)


def _sc_gather(table, idx, out_rows):
    t_rows, d = table.shape
    nw = 32
    rpw = out_rows // nw
    assert out_rows % nw == 0
    cap = max(8, (384 * 1024) // (d * 4) // 8 * 8)
    c_rows = min(rpw, cap)
    while rpw % c_rows:
        c_rows -= 8
    nchunks = rpw // c_rows
    idx_buf = max(16, c_rows)
    mesh = plsc.VectorSubcoreMesh(core_axis_name="c", subcore_axis_name="s")

    @functools.partial(
        pl.kernel, mesh=mesh,
        out_type=jax.ShapeDtypeStruct((out_rows, d), jnp.float32),
        scratch_types=[
            pltpu.VMEM((idx_buf,), jnp.int32),
            pltpu.VMEM((c_rows, d), jnp.float32),
            pltpu.SemaphoreType.DMA,
        ],
    )
    def k(table_hbm, idx_hbm, out_hbm, idx_v, rows_v, sem):
        wid = lax.axis_index("s") * 2 + lax.axis_index("c")
        for j in range(nchunks):
            base = wid * rpw + j * c_rows
            pltpu.sync_copy(idx_hbm.at[pl.ds(base, c_rows)],
                            idx_v.at[pl.ds(0, c_rows)])
            pltpu.async_copy(table_hbm.at[idx_v.at[pl.ds(0, c_rows)]],
                             rows_v, sem).wait()
            pltpu.sync_copy(rows_v, out_hbm.at[pl.ds(base, c_rows)])

    return k(table, idx)


def _pad_idx(perm, out_rows, zero_row):
    k = perm.shape[0]
    return jnp.concatenate(
        [perm, jnp.full((out_rows - k,), zero_row, jnp.int32)])


def _select(score, k):
    vals = jax.lax.top_k(score, k)[0]
    t = vals[k - 1]
    gt = score > t
    ngt = jnp.sum(gt.astype(jnp.int32))
    eq = score == t
    tie = eq & (jnp.cumsum(eq.astype(jnp.int32)) <= (k - ngt))
    sel = gt | tie
    rank = (jnp.cumsum(sel.astype(jnp.int32)) - 1).astype(jnp.int32)
    perm = jnp.nonzero(sel, size=k)[0].astype(jnp.int32)
    return sel, rank, perm, score[perm]


def _pad_rows(a, p):
    return jnp.pad(a, ((0, p - a.shape[0]), (0, 0)))


def _gather_factors(am, amt, perm, kp):
    k = perm.shape[0]
    psrc = am.shape[0]
    idx = _pad_idx(perm, kp, psrc - 1)
    u = _sc_gather(am, idx, kp)
    vt = _sc_gather(amt, idx, kp)
    return u, vt


def _unpool(res, hnext, sel, rank, out_rows):
    pn = hnext.shape[0]
    idx = jnp.where(sel, rank, pn - 1).astype(jnp.int32)
    idx = _pad_idx(idx, out_rows, pn - 1)
    up = _sc_gather(hnext, idx, out_rows)
    return res + up[: res.shape[0]]


def kernel(x, edge_index, p0, p1, p2, Wd0, Wd1, Wd2, bd0, bd1, bd2,
           Wu0, Wu1, bu0, bu1):
    n0, c = x.shape
    k1 = -(-n0 // 2)
    k2 = -(-k1 // 2)
    k3 = -(-k2 // 2)
    P0, P1, P2, P3 = (_rup(v, 256) for v in (n0, k1, k2, k3))

    src, dst = edge_index[0], edge_index[1]

    score0 = (x @ p0) / jnp.linalg.norm(p0)
    sel0, rank0, perm0, vals0 = _select(score0, k1)

    e_pad = _rup(edge_index.shape[1] + n0, 16 * 128 * 12)
    loops = jnp.arange(n0, dtype=jnp.int32)
    fill = jnp.full((e_pad - edge_index.shape[1] - n0,), n0, jnp.int32)
    srcp = jnp.concatenate([src, loops, fill])
    dstp = jnp.concatenate([dst, loops, fill])
    rank_tbl = jnp.where(sel0, rank0, -1).astype(jnp.int32)
    rank_tbl = jnp.pad(rank_tbl, (0, P0 - n0), constant_values=-1)
    u0f, vt0f = _sc_build_factors(srcp, dstp, rank_tbl, P1, P0)
    u0 = u0f.reshape(P1, P0)
    vt0 = vt0f.reshape(P1, P0)

    a1, a1t = _mm_nt(u0, vt0, k1)
    deg1 = _rowsum(a1, c)
    xp = jnp.pad(x, ((0, P0 - n0), (0, 0)))
    gate0 = jnp.pad(jnp.tanh(vals0), (0, P1 - k1))
    xg1 = _sc_gather(xp, _pad_idx(perm0, P1, P0 - 1), P1) * gate0[:, None]
    h1 = _agg(a1, _zscale(xg1, Wd0, deg1), deg1, bd0, k1)

    score1 = (h1[:k1] @ p1) / jnp.linalg.norm(p1)
    sel1, rank1, perm1, vals1 = _select(score1, k2)
    u1, vt1 = _gather_factors(a1, a1t, perm1, P2)
    a2, a2t = _mm_nt(u1, vt1, k2)
    deg2 = _rowsum(a2, c)
    gate1 = jnp.pad(jnp.tanh(vals1), (0, P2 - k2))
    xg2 = _sc_gather(h1, _pad_idx(perm1, P2, P1 - 1), P2) * gate1[:, None]
    h2 = _agg(a2, _zscale(xg2, Wd1, deg2), deg2, bd1, k2)

    score2 = (h2[:k2] @ p2) / jnp.linalg.norm(p2)
    sel2, rank2, perm2, vals2 = _select(score2, k3)
    u2, vt2 = _gather_factors(a2, a2t, perm2, P3)
    a3, _ = _mm_nt(u2, vt2, k3)
    deg3 = _rowsum(a3, c)
    gate2 = jnp.pad(jnp.tanh(vals2), (0, P3 - k3))
    xg3 = _sc_gather(h2, _pad_idx(perm2, P3, P2 - 1), P3) * gate2[:, None]
    h3 = _agg(a3, _zscale(xg3, Wd2, deg3), deg3, bd2, k3)

    r2 = _unpool(h2, h3, sel2, rank2, P2)
    g2 = _agg(a2, _zscale(r2, Wu0, deg2), deg2, bu0, k2)

    r1 = _unpool(h1, g2, sel1, rank1, P1)
    g1 = _agg(a1, _zscale(r1, Wu1, deg1), deg1, bu1, k1)

    return _unpool(x, g1, sel0, rank0, P0)[:n0]


# speedup vs baseline: 3.2794x; 1.0293x over previous
import functools
import math

import jax
import jax.numpy as jnp
from jax import lax
from jax.experimental import pallas as pl
from jax.experimental.pallas import tpu as pltpu
from jax.experimental.pallas import tpu_sc as plsc


def _rup(n, m):
    return ((n + m - 1) // m) * m


def _blk(p):
    return 512 if p % 512 == 0 else 256


def _mm_nt_body(u_ref, v_ref, o_ref, ot_ref, acc_ref, *, ksteps, n_real, bm,
                bn, want_t):
    @pl.when(pl.program_id(2) == 0)
    def _():
        acc_ref[...] = jnp.zeros_like(acc_ref)

    acc_ref[...] += jax.lax.dot_general(
        u_ref[...], v_ref[...], (((1,), (1,)), ((), ())),
        preferred_element_type=jnp.float32)

    @pl.when(pl.program_id(2) == ksteps - 1)
    def _():
        mi = pl.program_id(0)
        nj = pl.program_id(1)
        rows = mi * bm + jax.lax.broadcasted_iota(jnp.int32, (bm, bn), 0)
        cols = nj * bn + jax.lax.broadcasted_iota(jnp.int32, (bm, bn), 1)
        valid = (rows < n_real) & (cols < n_real)
        res = jnp.where(valid & (rows != cols), acc_ref[...], 0.0)
        res = res + jnp.where(valid & (rows == cols), 1.0, 0.0)
        o_ref[...] = res
        ot_ref[...] = res.T


def _mm_nt(u, vt, n_real):
    u = u.astype(jnp.bfloat16)
    vt = vt.astype(jnp.bfloat16)
    m, k = u.shape
    n = vt.shape[0]
    bm = 1024 if m % 1024 == 0 else _blk(m)
    bn = 1024 if n % 1024 == 0 else _blk(n)
    bk = 1024 if k % 1024 == 0 else _blk(k)
    grid = (m // bm, n // bn, k // bk)
    return pl.pallas_call(
        functools.partial(_mm_nt_body, ksteps=grid[2], n_real=n_real,
                          bm=bm, bn=bn, want_t=True),
        grid=grid,
        in_specs=[
            pl.BlockSpec((bm, bk), lambda i, j, kk: (i, kk)),
            pl.BlockSpec((bn, bk), lambda i, j, kk: (j, kk)),
        ],
        out_specs=[pl.BlockSpec((bm, bn), lambda i, j, kk: (i, j)),
                   pl.BlockSpec((bn, bm), lambda i, j, kk: (j, i))],
        out_shape=[jax.ShapeDtypeStruct((m, n), jnp.float32),
                   jax.ShapeDtypeStruct((n, m), jnp.float32)],
        scratch_shapes=[pltpu.VMEM((bm, bn), jnp.float32)],
        compiler_params=pltpu.CompilerParams(
            dimension_semantics=("parallel", "parallel", "arbitrary")),
    )(u, vt)


def _rowsum_body(a_ref, o_ref, *, nsteps, c):
    @pl.when(pl.program_id(1) == 0)
    def _():
        o_ref[...] = jnp.zeros_like(o_ref)

    s = jnp.sum(a_ref[...], axis=1, keepdims=True)
    o_ref[...] += jnp.broadcast_to(s, o_ref.shape)

    @pl.when(pl.program_id(1) == nsteps - 1)
    def _():
        o_ref[...] += 1.0


def _rowsum(a, c):
    m, n = a.shape
    bm, bn = _blk(m), _blk(n)
    grid = (m // bm, n // bn)
    return pl.pallas_call(
        functools.partial(_rowsum_body, nsteps=grid[1], c=c),
        grid=grid,
        in_specs=[pl.BlockSpec((bm, bn), lambda i, j: (i, j))],
        out_specs=pl.BlockSpec((bm, c), lambda i, j: (i, 0)),
        out_shape=jax.ShapeDtypeStruct((m, c), jnp.float32),
        compiler_params=pltpu.CompilerParams(
            dimension_semantics=("parallel", "arbitrary")),
    )(a)


def _zscale_body(inp_ref, w_ref, deg_ref, o_ref):
    xw = jnp.dot(inp_ref[...], w_ref[...], preferred_element_type=jnp.float32)
    o_ref[...] = xw * jax.lax.rsqrt(deg_ref[...])


def _zscale(inp, w, deg):
    m, c = inp.shape
    bm = _blk(m)
    grid = (m // bm,)
    return pl.pallas_call(
        _zscale_body,
        grid=grid,
        in_specs=[
            pl.BlockSpec((bm, c), lambda i: (i, 0)),
            pl.BlockSpec((c, c), lambda i: (0, 0)),
            pl.BlockSpec((bm, c), lambda i: (i, 0)),
        ],
        out_specs=pl.BlockSpec((bm, c), lambda i: (i, 0)),
        out_shape=jax.ShapeDtypeStruct((m, c), jnp.float32),
    )(inp, w, deg)


def _agg_body(a_ref, z_ref, zr_ref, deg_ref, b_ref, o_ref, acc_ref, *,
              ksteps, n_real, bm):
    @pl.when(pl.program_id(1) == 0)
    def _():
        acc_ref[...] = jnp.zeros_like(acc_ref)

    acc_ref[...] += jnp.dot(a_ref[...], z_ref[...],
                            preferred_element_type=jnp.float32)

    @pl.when(pl.program_id(1) == ksteps - 1)
    def _():
        dinv = jax.lax.rsqrt(deg_ref[...])
        h = dinv * (acc_ref[...] + zr_ref[...]) + b_ref[...]
        rows = pl.program_id(0) * bm + jax.lax.broadcasted_iota(
            jnp.int32, h.shape, 0)
        o_ref[...] = jnp.where(rows < n_real, jnp.maximum(h, 0.0), 0.0)


def _agg(a, z, deg, b, n_real):
    m, k = a.shape
    c = z.shape[1]
    bm = 1024 if m % 1024 == 0 else _blk(m)
    bk = 1024 if k % 1024 == 0 else _blk(k)
    grid = (m // bm, k // bk)
    return pl.pallas_call(
        functools.partial(_agg_body, ksteps=grid[1], n_real=n_real, bm=bm),
        grid=grid,
        in_specs=[
            pl.BlockSpec((bm, bk), lambda i, kk: (i, kk)),
            pl.BlockSpec((bk, c), lambda i, kk: (kk, 0)),
            pl.BlockSpec((bm, c), lambda i, kk: (i, 0)),
            pl.BlockSpec((bm, c), lambda i, kk: (i, 0)),
            pl.BlockSpec((1, c), lambda i, kk: (0, 0)),
        ],
        out_specs=pl.BlockSpec((bm, c), lambda i, kk: (i, 0)),
        out_shape=jax.ShapeDtypeStruct((m, c), jnp.float32),
        scratch_shapes=[pltpu.VMEM((bm, c), jnp.float32)],
        compiler_params=pltpu.CompilerParams(
            dimension_semantics=("parallel", "arbitrary")),
    )(a, z, z, deg, b.reshape(1, c))


_NBUF = 12
_IDXW = 128


def _sc_build_factors(src, dst, rank_tbl, kp, np_):
    e_tot = src.shape[0]
    n_tiles = 16
    eps = e_tot // n_tiles
    rows_dma = eps // _IDXW
    assert eps % (_IDXW * _NBUF) == 0
    groups = rows_dma // _NBUF
    rows_sc = kp // 2
    brows = 128
    while rows_sc % brows:
        brows -= 32
    nblocks = rows_sc // brows
    dump = brows * np_
    sh_words = dump + 32768
    stripe = (brows * np_) // n_tiles
    zch = stripe
    nzc = 1
    while zch > 12288:
        nzc *= 2
        zch = stripe // nzc
    assert stripe == zch * nzc
    tblsz = rank_tbl.shape[0]
    mesh = plsc.VectorSubcoreMesh(core_axis_name="c", subcore_axis_name="s")

    @functools.partial(
        pl.kernel, mesh=mesh,
        out_type=[jax.ShapeDtypeStruct((kp * np_,), jnp.float32),
                  jax.ShapeDtypeStruct((kp * np_,), jnp.float32)],
        scratch_types=[
            pltpu.VMEM((eps,), jnp.int32),
            pltpu.VMEM((eps,), jnp.int32),
            pltpu.VMEM((tblsz,), jnp.int32),
            pltpu.VMEM((_NBUF, _IDXW), jnp.int32),
            pltpu.VMEM((_IDXW,), jnp.float32),
            pltpu.VMEM((zch,), jnp.float32),
            pltpu.VMEM_SHARED((sh_words,), jnp.float32),
            pltpu.SemaphoreType.DMA,
        ],
        compiler_params=pltpu.CompilerParams(needs_layout_passes=False),
    )
    def k(src_hbm, dst_hbm, rank_hbm, u_hbm, vt_hbm, src_v, dst_v, rank_v,
          idx_v, val_v, zero_v, shared, sem):
        sc = lax.axis_index("c")
        tid = lax.axis_index("s")
        pltpu.sync_copy(src_hbm.at[pl.ds(tid * eps, eps)], src_v)
        pltpu.sync_copy(dst_hbm.at[pl.ds(tid * eps, eps)], dst_v)
        pltpu.sync_copy(rank_hbm, rank_v)

        def fillz(i, _):
            zero_v[pl.ds(i * 16, 16)] = jnp.zeros((16,), jnp.float32)
            return 0

        lax.fori_loop(0, zch // 16, fillz, 0)

        def fill1(i, _):
            val_v[pl.ds(i * 16, 16)] = jnp.ones((16,), jnp.float32)
            return 0

        lax.fori_loop(0, _IDXW // 16, fill1, 0)

        for out_hbm, key_v, col_v in ((u_hbm, src_v, dst_v),
                                      (vt_hbm, dst_v, src_v)):
            def block(b, _):
                r0 = sc * rows_sc + b * brows

                def zero(i, _):
                    pltpu.sync_copy(
                        zero_v,
                        shared.at[pl.ds(tid * stripe + i * zch, zch)])
                    return 0

                lax.fori_loop(0, nzc, zero, 0)
                plsc.subcore_barrier()

                def group(g, _):
                    handles = []
                    for j in range(_NBUF):
                        def chunk(ci, _, j=j, g=g):
                            off = (g * _NBUF + j) * _IDXW + ci * 16
                            keys = key_v[pl.ds(off, 16)]
                            cols = col_v[pl.ds(off, 16)]
                            rl = plsc.load_gather(rank_v, [keys]) - r0
                            inb = (rl >= 0) & (rl < brows)
                            idx_v[j, pl.ds(ci * 16, 16)] = jnp.where(
                                inb, rl * np_ + cols, dump + (keys & 32767))
                            return 0

                        lax.fori_loop(0, _IDXW // 16, chunk, 0)
                        handles.append(pltpu.async_copy(
                            val_v, shared.at[idx_v.at[j]], sem, add=True))
                    for h in handles:
                        h.wait()
                    return 0

                lax.fori_loop(0, groups, group, 0)
                plsc.subcore_barrier()
                pltpu.sync_copy(
                    shared.at[pl.ds(tid * stripe, stripe)],
                    out_hbm.at[pl.ds(r0 * np_ + tid * stripe, stripe)])
                plsc.subcore_barrier()
                return 0

            lax.fori_loop(0, nblocks, block, 0)

    return k(src, dst, rank_tbl)


def _sc_gather(table, idx, out_rows):
    t_rows, d = table.shape
    nw = 32
    rpw = out_rows // nw
    assert out_rows % nw == 0
    cap = max(8, (384 * 1024) // (d * 4) // 8 * 8)
    c_rows = min(rpw, cap)
    while rpw % c_rows:
        c_rows -= 8
    nchunks = rpw // c_rows
    idx_buf = max(16, c_rows)
    mesh = plsc.VectorSubcoreMesh(core_axis_name="c", subcore_axis_name="s")

    @functools.partial(
        pl.kernel, mesh=mesh,
        out_type=jax.ShapeDtypeStruct((out_rows, d), jnp.float32),
        scratch_types=[
            pltpu.VMEM((idx_buf,), jnp.int32),
            pltpu.VMEM((c_rows, d), jnp.float32),
            pltpu.SemaphoreType.DMA,
        ],
    )
    def k(table_hbm, idx_hbm, out_hbm, idx_v, rows_v, sem):
        wid = lax.axis_index("s") * 2 + lax.axis_index("c")
        for j in range(nchunks):
            base = wid * rpw + j * c_rows
            pltpu.sync_copy(idx_hbm.at[pl.ds(base, c_rows)],
                            idx_v.at[pl.ds(0, c_rows)])
            pltpu.async_copy(table_hbm.at[idx_v.at[pl.ds(0, c_rows)]],
                             rows_v, sem).wait()
            pltpu.sync_copy(rows_v, out_hbm.at[pl.ds(base, c_rows)])

    return k(table, idx)


def _pad_idx(perm, out_rows, zero_row):
    k = perm.shape[0]
    return jnp.concatenate(
        [perm, jnp.full((out_rows - k,), zero_row, jnp.int32)])


def _select(score, k):
    vals = jax.lax.top_k(score, k)[0]
    t = vals[k - 1]
    gt = score > t
    ngt = jnp.sum(gt.astype(jnp.int32))
    eq = score == t
    tie = eq & (jnp.cumsum(eq.astype(jnp.int32)) <= (k - ngt))
    sel = gt | tie
    rank = (jnp.cumsum(sel.astype(jnp.int32)) - 1).astype(jnp.int32)
    perm = jnp.nonzero(sel, size=k)[0].astype(jnp.int32)
    return sel, rank, perm, score[perm]


def _pad_rows(a, p):
    return jnp.pad(a, ((0, p - a.shape[0]), (0, 0)))


def _gather_factors(am, amt, perm, kp):
    k = perm.shape[0]
    psrc = am.shape[0]
    idx = _pad_idx(perm, kp, psrc - 1)
    u = _sc_gather(am, idx, kp)
    vt = _sc_gather(amt, idx, kp)
    return u, vt


def _unpool(res, hnext, sel, rank, out_rows):
    pn = hnext.shape[0]
    idx = jnp.where(sel, rank, pn - 1).astype(jnp.int32)
    idx = _pad_idx(idx, out_rows, pn - 1)
    up = _sc_gather(hnext, idx, out_rows)
    return res + up[: res.shape[0]]


def kernel(x, edge_index, p0, p1, p2, Wd0, Wd1, Wd2, bd0, bd1, bd2,
           Wu0, Wu1, bu0, bu1):
    n0, c = x.shape
    k1 = -(-n0 // 2)
    k2 = -(-k1 // 2)
    k3 = -(-k2 // 2)
    P0, P1, P2, P3 = (_rup(v, 256) for v in (n0, k1, k2, k3))

    src, dst = edge_index[0], edge_index[1]

    score0 = (x @ p0) / jnp.linalg.norm(p0)
    sel0, rank0, perm0, vals0 = _select(score0, k1)

    e_pad = _rup(edge_index.shape[1] + n0, 16 * 128 * 12)
    loops = jnp.arange(n0, dtype=jnp.int32)
    fill = jnp.full((e_pad - edge_index.shape[1] - n0,), n0, jnp.int32)
    srcp = jnp.concatenate([src, loops, fill])
    dstp = jnp.concatenate([dst, loops, fill])
    rank_tbl = jnp.where(sel0, rank0, -1).astype(jnp.int32)
    rank_tbl = jnp.pad(rank_tbl, (0, P0 - n0), constant_values=-1)
    u0f, vt0f = _sc_build_factors(srcp, dstp, rank_tbl, P1, P0)
    u0 = u0f.reshape(P1, P0)
    vt0 = vt0f.reshape(P1, P0)

    a1, a1t = _mm_nt(u0, vt0, k1)
    deg1 = _rowsum(a1, c)
    xp = jnp.pad(x, ((0, P0 - n0), (0, 0)))
    gate0 = jnp.pad(jnp.tanh(vals0), (0, P1 - k1))
    xg1 = _sc_gather(xp, _pad_idx(perm0, P1, P0 - 1), P1) * gate0[:, None]
    h1 = _agg(a1, _zscale(xg1, Wd0, deg1), deg1, bd0, k1)

    score1 = (h1[:k1] @ p1) / jnp.linalg.norm(p1)
    sel1, rank1, perm1, vals1 = _select(score1, k2)
    u1, vt1 = _gather_factors(a1, a1t, perm1, P2)
    a2, a2t = _mm_nt(u1, vt1, k2)
    deg2 = _rowsum(a2, c)
    gate1 = jnp.pad(jnp.tanh(vals1), (0, P2 - k2))
    xg2 = _sc_gather(h1, _pad_idx(perm1, P2, P1 - 1), P2) * gate1[:, None]
    h2 = _agg(a2, _zscale(xg2, Wd1, deg2), deg2, bd1, k2)

    score2 = (h2[:k2] @ p2) / jnp.linalg.norm(p2)
    sel2, rank2, perm2, vals2 = _select(score2, k3)
    u2, vt2 = _gather_factors(a2, a2t, perm2, P3)
    a3, _ = _mm_nt(u2, vt2, k3)
    deg3 = _rowsum(a3, c)
    gate2 = jnp.pad(jnp.tanh(vals2), (0, P3 - k3))
    xg3 = _sc_gather(h2, _pad_idx(perm2, P3, P2 - 1), P3) * gate2[:, None]
    h3 = _agg(a3, _zscale(xg3, Wd2, deg3), deg3, bd2, k3)

    r2 = _unpool(h2, h3, sel2, rank2, P2)
    g2 = _agg(a2, _zscale(r2, Wu0, deg2), deg2, bu0, k2)

    r1 = _unpool(h1, g2, sel1, rank1, P1)
    g1 = _agg(a1, _zscale(r1, Wu1, deg1), deg1, bu1, k1)

    return _unpool(x, g1, sel0, rank0, P0)[:n0]
